# Initial kernel scaffold; baseline (speedup 1.0000x reference)
import jax
import jax.numpy as jnp
from jax.experimental import pallas as pl


def kernel(nodes, senders, receivers, edge_attr, globals, batch, eW1, eb1, eW2, eb2, nW1, nb1, nW2, nb2, gW1, gb1, gW2, gb2):
    raise NotImplementedError("write your pallas kernel here")


# speedup vs baseline: 1.1948x; 1.1948x over previous
import functools

import jax
import jax.numpy as jnp
from jax import lax
from jax.experimental import pallas as pl
from jax.experimental.pallas import tpu as pltpu
from jax.experimental.pallas import tpu_sc as plsc

N = 10000
E = 160000
DF = 256
DE = 16
DEO = 64
DH = 512
DG = 64

NP = 10240
NP2 = 10112
NW = 32
EPW = E // NW
GK = 40
GC = EPW // GK
EPS = E // 16
SK = 80
SC_CHUNKS = EPS // SK

_mesh = plsc.VectorSubcoreMesh(core_axis_name="c", subcore_axis_name="s")


def _pre_body(x, wsc, wr, g, gw1, gb1, gw2, gb2, s_out, r_out, g_out):
    xb = x[:]
    s_out[:] = jnp.dot(xb, wsc[:], preferred_element_type=jnp.float32)
    r_out[:] = jnp.dot(xb, wr[:], preferred_element_type=jnp.float32)

    @pl.when(pl.program_id(0) == 0)
    def _():
        h = jnp.maximum(
            jnp.dot(g[:], gw1[:], preferred_element_type=jnp.float32) + gb1[:], 0.0)
        g_out[:] = jnp.dot(h, gw2[:], preferred_element_type=jnp.float32) + gb2[:]


def _pre_call(nodes_p, wsc, wr, g, gw1, gb1, gw2, gb2):
    full = lambda shape: pl.BlockSpec(shape, lambda i: (0, 0))
    return pl.pallas_call(
        _pre_body,
        grid=(NP // 256,),
        in_specs=[
            pl.BlockSpec((256, DF), lambda i: (i, 0)),
            full((DF, 2 * DH)),
            full((DF, DH)),
            full((1, DG)),
            full((DG, DG)),
            full((1, DG)),
            full((DG, DG)),
            full((1, DG)),
        ],
        out_specs=[
            pl.BlockSpec((256, 2 * DH), lambda i: (i, 0)),
            pl.BlockSpec((256, DH), lambda i: (i, 0)),
            full((1, DG)),
        ],
        out_shape=[
            jax.ShapeDtypeStruct((NP, 2 * DH), jnp.float32),
            jax.ShapeDtypeStruct((NP, DH), jnp.float32),
            jax.ShapeDtypeStruct((1, DG), jnp.float32),
        ],
    )(nodes_p, wsc, wr, g, gw1, gb1, gw2, gb2)


@functools.partial(
    pl.kernel,
    out_type=[
        jax.ShapeDtypeStruct((E, 2 * DH), jnp.float32),
        jax.ShapeDtypeStruct((E, DH), jnp.float32),
    ],
    mesh=_mesh,
    scratch_types=[
        pltpu.VMEM((GC, GK), jnp.int32),
        pltpu.VMEM((GC, GK), jnp.int32),
        pltpu.VMEM((GK, 2 * DH), jnp.float32),
        pltpu.VMEM((GK, DH), jnp.float32),
        pltpu.SemaphoreType.DMA,
        pltpu.SemaphoreType.DMA,
    ],
)
def _gather_kernel(s_tab, r_tab, senders3, receivers3,
                   gs_out, gr_out,
                   sidx, ridx, sbuf, rbuf, sem1, sem2):
    cid = lax.axis_index("c")
    sid = lax.axis_index("s")
    wid = sid * 2 + cid
    base = pl.multiple_of(wid * EPW, 8)
    pltpu.sync_copy(senders3.at[wid], sidx)
    pltpu.sync_copy(receivers3.at[wid], ridx)

    def body(i, carry):
        off = pl.multiple_of(i * GK, 8)
        cp1 = pltpu.async_copy(s_tab.at[sidx.at[i]], sbuf, sem1)
        cp2 = pltpu.async_copy(r_tab.at[ridx.at[i]], rbuf, sem2)
        cp1.wait()
        cp2.wait()
        pltpu.sync_copy(sbuf, gs_out.at[pl.ds(base + off, GK)])
        pltpu.sync_copy(rbuf, gr_out.at[pl.ds(base + off, GK)])
        return carry

    lax.fori_loop(0, GC, body, 0)


def _edge_body(gs, gr, ea, w1e, b1, w2, b2, nw1e, nb1, nw2, nb2,
               ne_out, ma_out, mb_out):
    a = gs[:, :DH]
    c = gs[:, DH:]
    h1 = jnp.maximum(
        a + gr[:] + jnp.dot(ea[:], w1e[:], preferred_element_type=jnp.float32)
        + b1[:], 0.0)
    ne = jnp.dot(h1, w2[:], preferred_element_type=jnp.float32) + b2[:]
    ne_out[:] = ne
    h2 = jnp.maximum(
        c + jnp.dot(ne, nw1e[:], preferred_element_type=jnp.float32) + nb1[:], 0.0)
    m = jnp.dot(h2, nw2[:], preferred_element_type=jnp.float32) + nb2[:]
    ma_out[:] = m[:, :128]
    mb_out[:] = m[:, 128:]


def _edge_call(gs, gr, ea, w1e, b1, w2, b2, nw1e, nb1, nw2, nb2):
    BE = 256
    full = lambda shape: pl.BlockSpec(shape, lambda i: (0, 0))
    return pl.pallas_call(
        _edge_body,
        grid=(E // BE,),
        in_specs=[
            pl.BlockSpec((BE, 2 * DH), lambda i: (i, 0)),
            pl.BlockSpec((BE, DH), lambda i: (i, 0)),
            pl.BlockSpec((BE, DE), lambda i: (i, 0)),
            full((DE, DH)),
            full((1, DH)),
            full((DH, DEO)),
            full((1, DEO)),
            full((DEO, DH)),
            full((1, DH)),
            full((DH, DF)),
            full((1, DF)),
        ],
        out_specs=[
            pl.BlockSpec((BE, DEO), lambda i: (i, 0)),
            pl.BlockSpec((BE, 128), lambda i: (i, 0)),
            pl.BlockSpec((BE, 128), lambda i: (i, 0)),
        ],
        out_shape=[
            jax.ShapeDtypeStruct((E, DEO), jnp.float32),
            jax.ShapeDtypeStruct((E, 128), jnp.float32),
            jax.ShapeDtypeStruct((E, 128), jnp.float32),
        ],
    )(gs, gr, ea, w1e, b1, w2, b2, nw1e, nb1, nw2, nb2)


@functools.partial(
    pl.kernel,
    out_type=[
        jax.ShapeDtypeStruct((NP2, 128), jnp.float32),
        jax.ShapeDtypeStruct((NP2, 128), jnp.float32),
        jax.ShapeDtypeStruct((NP2, 128), jnp.float32),
    ],
    mesh=_mesh,
    scratch_types=[
        pltpu.VMEM((SC_CHUNKS, SK), jnp.int32),
        pltpu.VMEM((SK, 128), jnp.float32),
        pltpu.VMEM((16, 128), jnp.float32),
        pltpu.VMEM((SK, 128), jnp.float32),
        pltpu.VMEM_SHARED((NP2, 128), jnp.float32),
    ],
)
def _scatter_kernel(msg_a, msg_b, receivers3,
                    sum_a_out, sum_b_out, cnt_out,
                    ridx, mbuf, zbuf, ones, acc):
    cid = lax.axis_index("c")
    sid = lax.axis_index("s")
    ebase = pl.multiple_of(sid * EPS, 8)
    pltpu.sync_copy(receivers3.at[sid], ridx)

    zero = jnp.zeros((16,), jnp.float32)
    one = jnp.ones((16,), jnp.float32)
    for r in range(16):
        for q in range(128 // 16):
            zbuf[r, pl.ds(q * 16, 16)] = zero
    for r in range(SK):
        for q in range(128 // 16):
            ones[r, pl.ds(q * 16, 16)] = one

    rows = NP2 // 16
    zb = sid * rows

    def zero_acc():
        for j in range(rows // 16):
            pltpu.sync_copy(zbuf, acc.at[pl.ds(zb + j * 16, 16)])
        pltpu.sync_copy(zbuf.at[pl.ds(0, 8)], acc.at[pl.ds(zb + rows - 8, 8)])

    zero_acc()
    plsc.subcore_barrier()

    def accumulate(msg_ref):
        def chunk(i, carry):
            off = pl.multiple_of(i * SK, 8)
            pltpu.sync_copy(msg_ref.at[pl.ds(ebase + off, SK)], mbuf)
            pltpu.sync_copy(mbuf, acc.at[ridx.at[i]], add=True)
            return carry
        lax.fori_loop(0, SC_CHUNKS, chunk, 0)

    @pl.when(cid == 0)
    def _():
        accumulate(msg_a)

    @pl.when(cid == 1)
    def _():
        accumulate(msg_b)

    plsc.subcore_barrier()

    @pl.when(cid == 0)
    def _():
        pltpu.sync_copy(acc.at[pl.ds(zb, rows)], sum_a_out.at[pl.ds(zb, rows)])

    @pl.when(cid == 1)
    def _():
        pltpu.sync_copy(acc.at[pl.ds(zb, rows)], sum_b_out.at[pl.ds(zb, rows)])

    @pl.when(cid == 0)
    def _():
        zero_acc()
        plsc.subcore_barrier()

        def cchunk(i, carry):
            pltpu.sync_copy(ones, acc.at[ridx.at[i]], add=True)
            return carry

        lax.fori_loop(0, SC_CHUNKS, cchunk, 0)
        plsc.subcore_barrier()
        pltpu.sync_copy(acc.at[pl.ds(zb, rows)], cnt_out.at[pl.ds(zb, rows)])


def _div_body(sa, sb, c, o):
    cnt = jnp.maximum(c[:, 0:1], 1.0)
    o[:] = jnp.concatenate([sa[:], sb[:]], axis=1) / cnt


def _div_call(sum_a, sum_b, cnt):
    BR = 128
    return pl.pallas_call(
        _div_body,
        grid=(NP2 // BR,),
        in_specs=[
            pl.BlockSpec((BR, 128), lambda i: (i, 0)),
            pl.BlockSpec((BR, 128), lambda i: (i, 0)),
            pl.BlockSpec((BR, 128), lambda i: (i, 0)),
        ],
        out_specs=pl.BlockSpec((BR, DF), lambda i: (i, 0)),
        out_shape=jax.ShapeDtypeStruct((NP2, DF), jnp.float32),
    )(sum_a, sum_b, cnt)


def kernel(nodes, senders, receivers, edge_attr, globals, batch,
           eW1, eb1, eW2, eb2, nW1, nb1, nW2, nb2, gW1, gb1, gW2, gb2):
    nodes_p = jnp.pad(nodes, ((0, NP - N), (0, 0)))
    wsc = jnp.concatenate([eW1[:DF], nW1[:DF]], axis=1)
    wr = eW1[DF:2 * DF]
    w1e = eW1[2 * DF:]
    nw1e = nW1[DF:]

    senders3 = senders.astype(jnp.int32).reshape(NW, GC, GK)
    receivers3 = receivers.astype(jnp.int32).reshape(NW, GC, GK)
    receivers3s = receivers.astype(jnp.int32).reshape(16, SC_CHUNKS, SK)

    s_tab, r_tab, g_out = _pre_call(
        nodes_p, wsc, wr, globals.reshape(1, DG),
        gW1, gb1.reshape(1, DG), gW2, gb2.reshape(1, DG))

    gs, gr = _gather_kernel(s_tab, r_tab, senders3, receivers3)

    new_edge, msg_a, msg_b = _edge_call(
        gs, gr, edge_attr, w1e, eb1.reshape(1, DH), eW2, eb2.reshape(1, DEO),
        nw1e, nb1.reshape(1, DH), nW2, nb2.reshape(1, DF))

    sum_a, sum_b, cnt = _scatter_kernel(msg_a, msg_b, receivers3s)
    new_nodes = _div_call(sum_a, sum_b, cnt)[:N]
    return (new_nodes, new_edge, g_out.reshape(DG))


# speedup vs baseline: 1.4480x; 1.2119x over previous
import functools

import jax
import jax.numpy as jnp
from jax import lax
from jax.experimental import pallas as pl
from jax.experimental.pallas import tpu as pltpu
from jax.experimental.pallas import tpu_sc as plsc

N = 10000
E = 160000
DF = 256
DE = 16
DEO = 64
DH = 512
DG = 64

NP = 10240
NP2 = 10112
K = 5
EC = E // K
NW = 32
EPW = EC // NW
GK = 40
GC = EPW // GK
EPS = EC // 16
SK = 80
SC_CHUNKS = EPS // SK
NC0 = 13

_mesh = plsc.VectorSubcoreMesh(core_axis_name="c", subcore_axis_name="s")


def _pre_body(x, wsc, wr, g, gw1, gb1, gw2, gb2, s_out, r_out, g_out):
    xb = x[:]
    s_out[:] = jnp.dot(xb, wsc[:], preferred_element_type=jnp.float32)
    r_out[:] = jnp.dot(xb, wr[:], preferred_element_type=jnp.float32)

    @pl.when(pl.program_id(0) == 0)
    def _():
        h = jnp.maximum(
            jnp.dot(g[:], gw1[:], preferred_element_type=jnp.float32) + gb1[:], 0.0)
        g_out[:] = jnp.dot(h, gw2[:], preferred_element_type=jnp.float32) + gb2[:]


def _pre_call(nodes_p, wsc, wr, g, gw1, gb1, gw2, gb2):
    full = lambda shape: pl.BlockSpec(shape, lambda i: (0, 0))
    return pl.pallas_call(
        _pre_body,
        grid=(NP // 256,),
        in_specs=[
            pl.BlockSpec((256, DF), lambda i: (i, 0)),
            full((DF, 2 * DH)),
            full((DF, DH)),
            full((1, DG)),
            full((DG, DG)),
            full((1, DG)),
            full((DG, DG)),
            full((1, DG)),
        ],
        out_specs=[
            pl.BlockSpec((256, 2 * DH), lambda i: (i, 0)),
            pl.BlockSpec((256, DH), lambda i: (i, 0)),
            full((1, DG)),
        ],
        out_shape=[
            jax.ShapeDtypeStruct((NP, 2 * DH), jnp.float32),
            jax.ShapeDtypeStruct((NP, DH), jnp.float32),
            jax.ShapeDtypeStruct((1, DG), jnp.float32),
        ],
    )(nodes_p, wsc, wr, g, gw1, gb1, gw2, gb2)


@functools.partial(
    pl.kernel,
    out_type=[
        jax.ShapeDtypeStruct((EC, 2 * DH), jnp.float32),
        jax.ShapeDtypeStruct((EC, DH), jnp.float32),
    ],
    mesh=_mesh,
    scratch_types=[
        pltpu.VMEM((GC, GK), jnp.int32),
        pltpu.VMEM((GC, GK), jnp.int32),
        pltpu.VMEM((GK, 2 * DH), jnp.float32),
        pltpu.VMEM((GK, DH), jnp.float32),
        pltpu.SemaphoreType.DMA,
        pltpu.SemaphoreType.DMA,
    ],
)
def _gather_kernel(s_tab, r_tab, senders3, receivers3,
                   gs_out, gr_out,
                   sidx, ridx, sbuf, rbuf, sem1, sem2):
    cid = lax.axis_index("c")
    sid = lax.axis_index("s")
    wid = sid * 2 + cid
    base = pl.multiple_of(wid * EPW, 8)
    pltpu.sync_copy(senders3.at[wid], sidx)
    pltpu.sync_copy(receivers3.at[wid], ridx)

    def body(i, carry):
        off = pl.multiple_of(i * GK, 8)
        cp1 = pltpu.async_copy(s_tab.at[sidx.at[i]], sbuf, sem1)
        cp2 = pltpu.async_copy(r_tab.at[ridx.at[i]], rbuf, sem2)
        cp1.wait()
        cp2.wait()
        pltpu.sync_copy(sbuf, gs_out.at[pl.ds(base + off, GK)])
        pltpu.sync_copy(rbuf, gr_out.at[pl.ds(base + off, GK)])
        return carry

    lax.fori_loop(0, GC, body, 0)


def _edge_body(gs, gr, ea, w1e, b1, w2, b2, nw1e, nb1, nw2, nb2,
               ne_out, ma_out, mb_out):
    a = gs[:, :DH]
    c = gs[:, DH:]
    h1 = jnp.maximum(
        a + gr[:] + jnp.dot(ea[:], w1e[:], preferred_element_type=jnp.float32)
        + b1[:], 0.0)
    ne = jnp.dot(h1, w2[:], preferred_element_type=jnp.float32) + b2[:]
    ne_out[:] = ne
    h2 = jnp.maximum(
        c + jnp.dot(ne, nw1e[:], preferred_element_type=jnp.float32) + nb1[:], 0.0)
    m = jnp.dot(h2, nw2[:], preferred_element_type=jnp.float32) + nb2[:]
    ma_out[:] = m[:, :128]
    mb_out[:] = m[:, 128:]


def _edge_call(gs, gr, ea, w1e, b1, w2, b2, nw1e, nb1, nw2, nb2):
    BE = 256
    full = lambda shape: pl.BlockSpec(shape, lambda i: (0, 0))
    return pl.pallas_call(
        _edge_body,
        grid=(EC // BE,),
        in_specs=[
            pl.BlockSpec((BE, 2 * DH), lambda i: (i, 0)),
            pl.BlockSpec((BE, DH), lambda i: (i, 0)),
            pl.BlockSpec((BE, DE), lambda i: (i, 0)),
            full((DE, DH)),
            full((1, DH)),
            full((DH, DEO)),
            full((1, DEO)),
            full((DEO, DH)),
            full((1, DH)),
            full((DH, DF)),
            full((1, DF)),
        ],
        out_specs=[
            pl.BlockSpec((BE, DEO), lambda i: (i, 0)),
            pl.BlockSpec((BE, 128), lambda i: (i, 0)),
            pl.BlockSpec((BE, 128), lambda i: (i, 0)),
        ],
        out_shape=[
            jax.ShapeDtypeStruct((EC, DEO), jnp.float32),
            jax.ShapeDtypeStruct((EC, 128), jnp.float32),
            jax.ShapeDtypeStruct((EC, 128), jnp.float32),
        ],
    )(gs, gr, ea, w1e, b1, w2, b2, nw1e, nb1, nw2, nb2)


@functools.partial(
    pl.kernel,
    out_type=[
        jax.ShapeDtypeStruct((NP2, 128), jnp.float32),
        jax.ShapeDtypeStruct((NP2, 128), jnp.float32),
        jax.ShapeDtypeStruct((NP2, 128), jnp.float32),
        jax.ShapeDtypeStruct((NP2, 128), jnp.float32),
    ],
    mesh=_mesh,
    scratch_types=[
        pltpu.VMEM((SC_CHUNKS, SK), jnp.int32),
        pltpu.VMEM((SK, 128), jnp.float32),
        pltpu.VMEM((16, 128), jnp.float32),
        pltpu.VMEM((SK, 128), jnp.float32),
        pltpu.VMEM_SHARED((NP2, 128), jnp.float32),
    ],
)
def _scatter_kernel(msg_a, msg_b, receivers3,
                    sum_a_out, sum_b_out, cnt_a_out, cnt_b_out,
                    ridx, mbuf, zbuf, ones, acc):
    cid = lax.axis_index("c")
    sid = lax.axis_index("s")
    ebase = pl.multiple_of(sid * EPS, 8)
    pltpu.sync_copy(receivers3.at[sid], ridx)

    zero = jnp.zeros((16,), jnp.float32)
    one = jnp.ones((16,), jnp.float32)
    for r in range(16):
        for q in range(128 // 16):
            zbuf[r, pl.ds(q * 16, 16)] = zero
    for r in range(SK):
        for q in range(128 // 16):
            ones[r, pl.ds(q * 16, 16)] = one

    rows = NP2 // 16
    zb = sid * rows

    def zero_acc():
        for j in range(rows // 16):
            pltpu.sync_copy(zbuf, acc.at[pl.ds(zb + j * 16, 16)])
        pltpu.sync_copy(zbuf.at[pl.ds(0, 8)], acc.at[pl.ds(zb + rows - 8, 8)])

    zero_acc()
    plsc.subcore_barrier()

    def accumulate(msg_ref):
        def chunk(i, carry):
            off = pl.multiple_of(i * SK, 8)
            pltpu.sync_copy(msg_ref.at[pl.ds(ebase + off, SK)], mbuf)
            pltpu.sync_copy(mbuf, acc.at[ridx.at[i]], add=True)
            return carry
        lax.fori_loop(0, SC_CHUNKS, chunk, 0)

    @pl.when(cid == 0)
    def _():
        accumulate(msg_a)

    @pl.when(cid == 1)
    def _():
        accumulate(msg_b)

    plsc.subcore_barrier()

    @pl.when(cid == 0)
    def _():
        pltpu.sync_copy(acc.at[pl.ds(zb, rows)], sum_a_out.at[pl.ds(zb, rows)])

    @pl.when(cid == 1)
    def _():
        pltpu.sync_copy(acc.at[pl.ds(zb, rows)], sum_b_out.at[pl.ds(zb, rows)])

    zero_acc()
    plsc.subcore_barrier()

    def cchunk(i, carry):
        pltpu.sync_copy(ones, acc.at[ridx.at[i]], add=True)
        return carry

    @pl.when(cid == 0)
    def _():
        lax.fori_loop(0, NC0, cchunk, 0)

    @pl.when(cid == 1)
    def _():
        lax.fori_loop(NC0, SC_CHUNKS, cchunk, 0)

    plsc.subcore_barrier()

    @pl.when(cid == 0)
    def _():
        pltpu.sync_copy(acc.at[pl.ds(zb, rows)], cnt_a_out.at[pl.ds(zb, rows)])

    @pl.when(cid == 1)
    def _():
        pltpu.sync_copy(acc.at[pl.ds(zb, rows)], cnt_b_out.at[pl.ds(zb, rows)])


def _div_body(*refs):
    sa_refs = refs[:K]
    sb_refs = refs[K:2 * K]
    cn_refs = refs[2 * K:4 * K]
    o = refs[4 * K]
    sa = sa_refs[0][:]
    sb = sb_refs[0][:]
    for r in sa_refs[1:]:
        sa = sa + r[:]
    for r in sb_refs[1:]:
        sb = sb + r[:]
    cnt = cn_refs[0][:, 0:1]
    for r in cn_refs[1:]:
        cnt = cnt + r[:, 0:1]
    o[:] = jnp.concatenate([sa, sb], axis=1) / jnp.maximum(cnt, 1.0)


def _div_call(sum_as, sum_bs, cnts):
    BR = 128
    block = pl.BlockSpec((BR, 128), lambda i: (i, 0))
    n_in = 4 * K
    return pl.pallas_call(
        _div_body,
        grid=(NP2 // BR,),
        in_specs=[block] * n_in,
        out_specs=pl.BlockSpec((BR, DF), lambda i: (i, 0)),
        out_shape=jax.ShapeDtypeStruct((NP2, DF), jnp.float32),
    )(*sum_as, *sum_bs, *cnts)


def kernel(nodes, senders, receivers, edge_attr, globals, batch,
           eW1, eb1, eW2, eb2, nW1, nb1, nW2, nb2, gW1, gb1, gW2, gb2):
    nodes_p = jnp.pad(nodes, ((0, NP - N), (0, 0)))
    wsc = jnp.concatenate([eW1[:DF], nW1[:DF]], axis=1)
    wr = eW1[DF:2 * DF]
    w1e = eW1[2 * DF:]
    nw1e = nW1[DF:]

    senders4 = senders.astype(jnp.int32).reshape(K, NW, GC, GK)
    receivers4 = receivers.astype(jnp.int32).reshape(K, NW, GC, GK)
    receivers4s = receivers.astype(jnp.int32).reshape(K, 16, SC_CHUNKS, SK)

    s_tab, r_tab, g_out = _pre_call(
        nodes_p, wsc, wr, globals.reshape(1, DG),
        gW1, gb1.reshape(1, DG), gW2, gb2.reshape(1, DG))

    eb1r = eb1.reshape(1, DH)
    eb2r = eb2.reshape(1, DEO)
    nb1r = nb1.reshape(1, DH)
    nb2r = nb2.reshape(1, DF)

    ne_parts, sum_as, sum_bs, cnts = [], [], [], []
    for k in range(K):
        gs, gr = _gather_kernel(s_tab, r_tab, senders4[k], receivers4[k])
        ne_k, ma_k, mb_k = _edge_call(
            gs, gr, edge_attr[k * EC:(k + 1) * EC],
            w1e, eb1r, eW2, eb2r, nw1e, nb1r, nW2, nb2r)
        sa_k, sb_k, ca_k, cb_k = _scatter_kernel(ma_k, mb_k, receivers4s[k])
        ne_parts.append(ne_k)
        sum_as.append(sa_k)
        sum_bs.append(sb_k)
        cnts.append(ca_k)
        cnts.append(cb_k)

    new_nodes = _div_call(sum_as, sum_bs, cnts)[:N]
    new_edge = jnp.concatenate(ne_parts, axis=0)
    return (new_nodes, new_edge, g_out.reshape(DG))


# speedup vs baseline: 1.9206x; 1.3264x over previous
import functools

import jax
import jax.numpy as jnp
from jax import lax
from jax.experimental import pallas as pl
from jax.experimental.pallas import tpu as pltpu
from jax.experimental.pallas import tpu_sc as plsc

N = 10000
E = 160000
DF = 256
DE = 16
DEO = 64
DH = 512
DG = 64

NP = 10240
NP2 = 10112
K = 5
EC = E // K
NW = 32
EPW = EC // NW
GK = 40
GC = EPW // GK
EPS = EC // 16
SK = 80
SC_CHUNKS = EPS // SK
NC0 = 13

_mesh = plsc.VectorSubcoreMesh(core_axis_name="c", subcore_axis_name="s")

_HI = -65536


def _pack(lo_f32, hi_f32):
    lo = lo_f32.astype(jnp.bfloat16).astype(jnp.float32)
    hi = hi_f32.astype(jnp.bfloat16).astype(jnp.float32)
    lo_b = lax.bitcast_convert_type(lo, jnp.int32)
    hi_b = lax.bitcast_convert_type(hi, jnp.int32)
    return (hi_b & _HI) | lax.shift_right_logical(lo_b, 16)


def _unpack_lo(w):
    return lax.bitcast_convert_type(lax.shift_left(w, 16), jnp.float32)


def _unpack_hi(w):
    return lax.bitcast_convert_type(w & _HI, jnp.float32)


def _pre_body(x, wsc, wr, g, gw1, gb1, gw2, gb2, s_out, r_out, g_out):
    xb = x[:]
    ac = jnp.dot(xb, wsc[:], preferred_element_type=jnp.float32)
    s_out[:] = _pack(ac[:, :DH], ac[:, DH:])
    b = jnp.dot(xb, wr[:], preferred_element_type=jnp.float32)
    r_out[:] = _pack(b[:, :DH // 2], b[:, DH // 2:])

    @pl.when(pl.program_id(0) == 0)
    def _():
        h = jnp.maximum(
            jnp.dot(g[:], gw1[:], preferred_element_type=jnp.float32) + gb1[:], 0.0)
        g_out[:] = jnp.dot(h, gw2[:], preferred_element_type=jnp.float32) + gb2[:]


def _pre_call(nodes_p, wsc, wr, g, gw1, gb1, gw2, gb2):
    full = lambda shape: pl.BlockSpec(shape, lambda i: (0, 0))
    return pl.pallas_call(
        _pre_body,
        grid=(NP // 256,),
        in_specs=[
            pl.BlockSpec((256, DF), lambda i: (i, 0)),
            full((DF, 2 * DH)),
            full((DF, DH)),
            full((1, DG)),
            full((DG, DG)),
            full((1, DG)),
            full((DG, DG)),
            full((1, DG)),
        ],
        out_specs=[
            pl.BlockSpec((256, DH), lambda i: (i, 0)),
            pl.BlockSpec((256, DH // 2), lambda i: (i, 0)),
            full((1, DG)),
        ],
        out_shape=[
            jax.ShapeDtypeStruct((NP, DH), jnp.int32),
            jax.ShapeDtypeStruct((NP, DH // 2), jnp.int32),
            jax.ShapeDtypeStruct((1, DG), jnp.float32),
        ],
    )(nodes_p, wsc, wr, g, gw1, gb1, gw2, gb2)


@functools.partial(
    pl.kernel,
    out_type=[
        jax.ShapeDtypeStruct((EC, DH), jnp.int32),
        jax.ShapeDtypeStruct((EC, DH // 2), jnp.int32),
    ],
    mesh=_mesh,
    scratch_types=[
        pltpu.VMEM((GC, GK), jnp.int32),
        pltpu.VMEM((GC, GK), jnp.int32),
        pltpu.VMEM((GK, DH), jnp.int32),
        pltpu.VMEM((GK, DH // 2), jnp.int32),
        pltpu.SemaphoreType.DMA,
        pltpu.SemaphoreType.DMA,
    ],
)
def _gather_kernel(s_tab, r_tab, senders3, receivers3,
                   gs_out, gr_out,
                   sidx, ridx, sbuf, rbuf, sem1, sem2):
    cid = lax.axis_index("c")
    sid = lax.axis_index("s")
    wid = sid * 2 + cid
    base = pl.multiple_of(wid * EPW, 8)
    pltpu.sync_copy(senders3.at[wid], sidx)
    pltpu.sync_copy(receivers3.at[wid], ridx)

    def body(i, carry):
        off = pl.multiple_of(i * GK, 8)
        cp1 = pltpu.async_copy(s_tab.at[sidx.at[i]], sbuf, sem1)
        cp2 = pltpu.async_copy(r_tab.at[ridx.at[i]], rbuf, sem2)
        cp1.wait()
        cp2.wait()
        pltpu.sync_copy(sbuf, gs_out.at[pl.ds(base + off, GK)])
        pltpu.sync_copy(rbuf, gr_out.at[pl.ds(base + off, GK)])
        return carry

    lax.fori_loop(0, GC, body, 0)


def _edge_body(gs, gr, ea, w1e, b1, w2, b2, nw1e, nb1, nw2, nb2,
               ne_out, ma_out, mb_out):
    w = gs[:]
    a = _unpack_lo(w)
    c = _unpack_hi(w)
    wrw = gr[:]
    b_full = jnp.concatenate([_unpack_lo(wrw), _unpack_hi(wrw)], axis=1)
    h1 = jnp.maximum(
        a + b_full
        + jnp.dot(ea[:], w1e[:], preferred_element_type=jnp.float32)
        + b1[:], 0.0)
    ne = jnp.dot(h1, w2[:], preferred_element_type=jnp.float32) + b2[:]
    ne_out[:] = ne
    h2 = jnp.maximum(
        c + jnp.dot(ne, nw1e[:], preferred_element_type=jnp.float32) + nb1[:], 0.0)
    m = jnp.dot(h2, nw2[:], preferred_element_type=jnp.float32) + nb2[:]
    ma_out[:] = m[:, :128]
    mb_out[:] = m[:, 128:]


def _edge_call(gs, gr, ea, w1e, b1, w2, b2, nw1e, nb1, nw2, nb2):
    BE = 256
    full = lambda shape: pl.BlockSpec(shape, lambda i: (0, 0))
    return pl.pallas_call(
        _edge_body,
        grid=(EC // BE,),
        in_specs=[
            pl.BlockSpec((BE, DH), lambda i: (i, 0)),
            pl.BlockSpec((BE, DH // 2), lambda i: (i, 0)),
            pl.BlockSpec((BE, DE), lambda i: (i, 0)),
            full((DE, DH)),
            full((1, DH)),
            full((DH, DEO)),
            full((1, DEO)),
            full((DEO, DH)),
            full((1, DH)),
            full((DH, DF)),
            full((1, DF)),
        ],
        out_specs=[
            pl.BlockSpec((BE, DEO), lambda i: (i, 0)),
            pl.BlockSpec((BE, 128), lambda i: (i, 0)),
            pl.BlockSpec((BE, 128), lambda i: (i, 0)),
        ],
        out_shape=[
            jax.ShapeDtypeStruct((EC, DEO), jnp.float32),
            jax.ShapeDtypeStruct((EC, 128), jnp.float32),
            jax.ShapeDtypeStruct((EC, 128), jnp.float32),
        ],
    )(gs, gr, ea, w1e, b1, w2, b2, nw1e, nb1, nw2, nb2)


@functools.partial(
    pl.kernel,
    out_type=[
        jax.ShapeDtypeStruct((NP2, 128), jnp.float32),
        jax.ShapeDtypeStruct((NP2, 128), jnp.float32),
        jax.ShapeDtypeStruct((NP2, 128), jnp.float32),
        jax.ShapeDtypeStruct((NP2, 128), jnp.float32),
    ],
    mesh=_mesh,
    scratch_types=[
        pltpu.VMEM((SC_CHUNKS, SK), jnp.int32),
        pltpu.VMEM((SK, 128), jnp.float32),
        pltpu.VMEM((16, 128), jnp.float32),
        pltpu.VMEM((SK, 128), jnp.float32),
        pltpu.VMEM_SHARED((NP2, 128), jnp.float32),
    ],
)
def _scatter_kernel(msg_a, msg_b, receivers3,
                    sum_a_out, sum_b_out, cnt_a_out, cnt_b_out,
                    ridx, mbuf, zbuf, ones, acc):
    cid = lax.axis_index("c")
    sid = lax.axis_index("s")
    ebase = pl.multiple_of(sid * EPS, 8)
    pltpu.sync_copy(receivers3.at[sid], ridx)

    zero = jnp.zeros((16,), jnp.float32)
    one = jnp.ones((16,), jnp.float32)
    for r in range(16):
        for q in range(128 // 16):
            zbuf[r, pl.ds(q * 16, 16)] = zero
    for r in range(SK):
        for q in range(128 // 16):
            ones[r, pl.ds(q * 16, 16)] = one

    rows = NP2 // 16
    zb = sid * rows

    def zero_acc():
        for j in range(rows // 16):
            pltpu.sync_copy(zbuf, acc.at[pl.ds(zb + j * 16, 16)])
        pltpu.sync_copy(zbuf.at[pl.ds(0, 8)], acc.at[pl.ds(zb + rows - 8, 8)])

    zero_acc()
    plsc.subcore_barrier()

    def accumulate(msg_ref):
        def chunk(i, carry):
            off = pl.multiple_of(i * SK, 8)
            pltpu.sync_copy(msg_ref.at[pl.ds(ebase + off, SK)], mbuf)
            pltpu.sync_copy(mbuf, acc.at[ridx.at[i]], add=True)
            return carry
        lax.fori_loop(0, SC_CHUNKS, chunk, 0)

    @pl.when(cid == 0)
    def _():
        accumulate(msg_a)

    @pl.when(cid == 1)
    def _():
        accumulate(msg_b)

    plsc.subcore_barrier()

    @pl.when(cid == 0)
    def _():
        pltpu.sync_copy(acc.at[pl.ds(zb, rows)], sum_a_out.at[pl.ds(zb, rows)])

    @pl.when(cid == 1)
    def _():
        pltpu.sync_copy(acc.at[pl.ds(zb, rows)], sum_b_out.at[pl.ds(zb, rows)])

    zero_acc()
    plsc.subcore_barrier()

    def cchunk(i, carry):
        pltpu.sync_copy(ones, acc.at[ridx.at[i]], add=True)
        return carry

    @pl.when(cid == 0)
    def _():
        lax.fori_loop(0, NC0, cchunk, 0)

    @pl.when(cid == 1)
    def _():
        lax.fori_loop(NC0, SC_CHUNKS, cchunk, 0)

    plsc.subcore_barrier()

    @pl.when(cid == 0)
    def _():
        pltpu.sync_copy(acc.at[pl.ds(zb, rows)], cnt_a_out.at[pl.ds(zb, rows)])

    @pl.when(cid == 1)
    def _():
        pltpu.sync_copy(acc.at[pl.ds(zb, rows)], cnt_b_out.at[pl.ds(zb, rows)])


def _div_body(*refs):
    sa_refs = refs[:K]
    sb_refs = refs[K:2 * K]
    cn_refs = refs[2 * K:4 * K]
    o = refs[4 * K]
    sa = sa_refs[0][:]
    sb = sb_refs[0][:]
    for r in sa_refs[1:]:
        sa = sa + r[:]
    for r in sb_refs[1:]:
        sb = sb + r[:]
    cnt = cn_refs[0][:, 0:1]
    for r in cn_refs[1:]:
        cnt = cnt + r[:, 0:1]
    o[:] = jnp.concatenate([sa, sb], axis=1) / jnp.maximum(cnt, 1.0)


def _div_call(sum_as, sum_bs, cnts):
    BR = 128
    block = pl.BlockSpec((BR, 128), lambda i: (i, 0))
    n_in = 4 * K
    return pl.pallas_call(
        _div_body,
        grid=(NP2 // BR,),
        in_specs=[block] * n_in,
        out_specs=pl.BlockSpec((BR, DF), lambda i: (i, 0)),
        out_shape=jax.ShapeDtypeStruct((NP2, DF), jnp.float32),
    )(*sum_as, *sum_bs, *cnts)


def kernel(nodes, senders, receivers, edge_attr, globals, batch,
           eW1, eb1, eW2, eb2, nW1, nb1, nW2, nb2, gW1, gb1, gW2, gb2):
    nodes_p = jnp.pad(nodes, ((0, NP - N), (0, 0)))
    wsc = jnp.concatenate([eW1[:DF], nW1[:DF]], axis=1)
    wr = eW1[DF:2 * DF]
    w1e = eW1[2 * DF:]
    nw1e = nW1[DF:]

    senders4 = senders.astype(jnp.int32).reshape(K, NW, GC, GK)
    receivers4 = receivers.astype(jnp.int32).reshape(K, NW, GC, GK)
    receivers4s = receivers.astype(jnp.int32).reshape(K, 16, SC_CHUNKS, SK)

    s_tab, r_tab, g_out = _pre_call(
        nodes_p, wsc, wr, globals.reshape(1, DG),
        gW1, gb1.reshape(1, DG), gW2, gb2.reshape(1, DG))

    eb1r = eb1.reshape(1, DH)
    eb2r = eb2.reshape(1, DEO)
    nb1r = nb1.reshape(1, DH)
    nb2r = nb2.reshape(1, DF)

    ne_parts, sum_as, sum_bs, cnts = [], [], [], []
    for k in range(K):
        gs, gr = _gather_kernel(s_tab, r_tab, senders4[k], receivers4[k])
        ne_k, ma_k, mb_k = _edge_call(
            gs, gr, edge_attr[k * EC:(k + 1) * EC],
            w1e, eb1r, eW2, eb2r, nw1e, nb1r, nW2, nb2r)
        sa_k, sb_k, ca_k, cb_k = _scatter_kernel(ma_k, mb_k, receivers4s[k])
        ne_parts.append(ne_k)
        sum_as.append(sa_k)
        sum_bs.append(sb_k)
        cnts.append(ca_k)
        cnts.append(cb_k)

    new_nodes = _div_call(sum_as, sum_bs, cnts)[:N]
    new_edge = jnp.concatenate(ne_parts, axis=0)
    return (new_nodes, new_edge, g_out.reshape(DG))


# speedup vs baseline: 2.1216x; 1.1047x over previous
import functools

import jax
import jax.numpy as jnp
from jax import lax
from jax.experimental import pallas as pl
from jax.experimental.pallas import tpu as pltpu
from jax.experimental.pallas import tpu_sc as plsc

N = 10000
E = 160000
DF = 256
DE = 16
DEO = 64
DH = 512
DG = 64

NP = 10240
NP2 = 10112
K = 5
EC = E // K
NW = 32
EPW = EC // NW
GK = 40
GC = EPW // GK
EPS = EC // 16
SK = 80
SC_CHUNKS = EPS // SK
NC0 = 13

_mesh = plsc.VectorSubcoreMesh(core_axis_name="c", subcore_axis_name="s")

_HI = -65536


def _pack(lo_f32, hi_f32):
    lo = lo_f32.astype(jnp.bfloat16).astype(jnp.float32)
    hi = hi_f32.astype(jnp.bfloat16).astype(jnp.float32)
    lo_b = lax.bitcast_convert_type(lo, jnp.int32)
    hi_b = lax.bitcast_convert_type(hi, jnp.int32)
    return (hi_b & _HI) | lax.shift_right_logical(lo_b, 16)


def _unpack_lo(w):
    return lax.bitcast_convert_type(lax.shift_left(w, 16), jnp.float32)


def _unpack_hi(w):
    return lax.bitcast_convert_type(w & _HI, jnp.float32)


def _pre_body(x, g, gw1, gb1, gw2, gb2, pack_out, g_out):
    xb = x[:]
    pack_out[:] = _pack(xb[:, :128], xb[:, 128:])

    @pl.when(pl.program_id(0) == 0)
    def _():
        h = jnp.maximum(
            jnp.dot(g[:], gw1[:], preferred_element_type=jnp.float32) + gb1[:], 0.0)
        g_out[:] = jnp.dot(h, gw2[:], preferred_element_type=jnp.float32) + gb2[:]


def _pre_call(nodes_p, g, gw1, gb1, gw2, gb2):
    full = lambda shape: pl.BlockSpec(shape, lambda i: (0, 0))
    return pl.pallas_call(
        _pre_body,
        grid=(NP // 256,),
        in_specs=[
            pl.BlockSpec((256, DF), lambda i: (i, 0)),
            full((1, DG)),
            full((DG, DG)),
            full((1, DG)),
            full((DG, DG)),
            full((1, DG)),
        ],
        out_specs=[
            pl.BlockSpec((256, 128), lambda i: (i, 0)),
            full((1, DG)),
        ],
        out_shape=[
            jax.ShapeDtypeStruct((NP, 128), jnp.int32),
            jax.ShapeDtypeStruct((1, DG), jnp.float32),
        ],
    )(nodes_p, g, gw1, gb1, gw2, gb2)


@functools.partial(
    pl.kernel,
    out_type=[
        jax.ShapeDtypeStruct((EC, 128), jnp.int32),
        jax.ShapeDtypeStruct((EC, 128), jnp.int32),
    ],
    mesh=_mesh,
    scratch_types=[
        pltpu.VMEM((GC, GK), jnp.int32),
        pltpu.VMEM((GC, GK), jnp.int32),
        pltpu.VMEM((GK, 128), jnp.int32),
        pltpu.VMEM((GK, 128), jnp.int32),
        pltpu.SemaphoreType.DMA,
        pltpu.SemaphoreType.DMA,
    ],
)
def _gather_kernel(tab, senders3, receivers3,
                   gs_out, gr_out,
                   sidx, ridx, sbuf, rbuf, sem1, sem2):
    cid = lax.axis_index("c")
    sid = lax.axis_index("s")
    wid = sid * 2 + cid
    base = pl.multiple_of(wid * EPW, 8)
    pltpu.sync_copy(senders3.at[wid], sidx)
    pltpu.sync_copy(receivers3.at[wid], ridx)

    def body(i, carry):
        off = pl.multiple_of(i * GK, 8)
        cp1 = pltpu.async_copy(tab.at[sidx.at[i]], sbuf, sem1)
        cp2 = pltpu.async_copy(tab.at[ridx.at[i]], rbuf, sem2)
        cp1.wait()
        cp2.wait()
        pltpu.sync_copy(sbuf, gs_out.at[pl.ds(base + off, GK)])
        pltpu.sync_copy(rbuf, gr_out.at[pl.ds(base + off, GK)])
        return carry

    lax.fori_loop(0, GC, body, 0)


def _edge_body(gs, gr, ea, wsr, w1e, b1, w2, b2, nw1s, nw1e, nb1, nw2, nb2,
               ne_out, ma_out, mb_out):
    ws = gs[:]
    s_b = jnp.concatenate(
        [_unpack_lo(ws), _unpack_hi(ws)], axis=1).astype(jnp.bfloat16)
    wr = gr[:]
    r_b = jnp.concatenate(
        [_unpack_lo(wr), _unpack_hi(wr)], axis=1).astype(jnp.bfloat16)
    x = jnp.concatenate([s_b, r_b], axis=1)
    h1 = jnp.maximum(
        jnp.dot(x, wsr[:], preferred_element_type=jnp.float32)
        + jnp.dot(ea[:].astype(jnp.bfloat16), w1e[:],
                  preferred_element_type=jnp.float32)
        + b1[:], 0.0)
    ne = jnp.dot(h1.astype(jnp.bfloat16), w2[:],
                 preferred_element_type=jnp.float32) + b2[:]
    ne_out[:] = ne
    h2 = jnp.maximum(
        jnp.dot(s_b, nw1s[:], preferred_element_type=jnp.float32)
        + jnp.dot(ne.astype(jnp.bfloat16), nw1e[:],
                  preferred_element_type=jnp.float32)
        + nb1[:], 0.0)
    m = jnp.dot(h2.astype(jnp.bfloat16), nw2[:],
                preferred_element_type=jnp.float32) + nb2[:]
    ma_out[:] = m[:, :128]
    mb_out[:] = m[:, 128:]


def _edge_call(gs, gr, ea, wsr, w1e, b1, w2, b2, nw1s, nw1e, nb1, nw2, nb2):
    BE = 256
    full = lambda shape: pl.BlockSpec(shape, lambda i: (0, 0))
    return pl.pallas_call(
        _edge_body,
        grid=(EC // BE,),
        in_specs=[
            pl.BlockSpec((BE, 128), lambda i: (i, 0)),
            pl.BlockSpec((BE, 128), lambda i: (i, 0)),
            pl.BlockSpec((BE, DE), lambda i: (i, 0)),
            full((2 * DF, DH)),
            full((DE, DH)),
            full((1, DH)),
            full((DH, DEO)),
            full((1, DEO)),
            full((DF, DH)),
            full((DEO, DH)),
            full((1, DH)),
            full((DH, DF)),
            full((1, DF)),
        ],
        out_specs=[
            pl.BlockSpec((BE, DEO), lambda i: (i, 0)),
            pl.BlockSpec((BE, 128), lambda i: (i, 0)),
            pl.BlockSpec((BE, 128), lambda i: (i, 0)),
        ],
        out_shape=[
            jax.ShapeDtypeStruct((EC, DEO), jnp.float32),
            jax.ShapeDtypeStruct((EC, 128), jnp.float32),
            jax.ShapeDtypeStruct((EC, 128), jnp.float32),
        ],
    )(gs, gr, ea, wsr, w1e, b1, w2, b2, nw1s, nw1e, nb1, nw2, nb2)


@functools.partial(
    pl.kernel,
    out_type=[
        jax.ShapeDtypeStruct((NP2, 128), jnp.float32),
        jax.ShapeDtypeStruct((NP2, 128), jnp.float32),
        jax.ShapeDtypeStruct((NP2, 128), jnp.float32),
        jax.ShapeDtypeStruct((NP2, 128), jnp.float32),
    ],
    mesh=_mesh,
    scratch_types=[
        pltpu.VMEM((SC_CHUNKS, SK), jnp.int32),
        pltpu.VMEM((SK, 128), jnp.float32),
        pltpu.VMEM((16, 128), jnp.float32),
        pltpu.VMEM((SK, 128), jnp.float32),
        pltpu.VMEM_SHARED((NP2, 128), jnp.float32),
    ],
)
def _scatter_kernel(msg_a, msg_b, receivers3,
                    sum_a_out, sum_b_out, cnt_a_out, cnt_b_out,
                    ridx, mbuf, zbuf, ones, acc):
    cid = lax.axis_index("c")
    sid = lax.axis_index("s")
    ebase = pl.multiple_of(sid * EPS, 8)
    pltpu.sync_copy(receivers3.at[sid], ridx)

    zero = jnp.zeros((16,), jnp.float32)
    one = jnp.ones((16,), jnp.float32)
    for r in range(16):
        for q in range(128 // 16):
            zbuf[r, pl.ds(q * 16, 16)] = zero
    for r in range(SK):
        for q in range(128 // 16):
            ones[r, pl.ds(q * 16, 16)] = one

    rows = NP2 // 16
    zb = sid * rows

    def zero_acc():
        for j in range(rows // 16):
            pltpu.sync_copy(zbuf, acc.at[pl.ds(zb + j * 16, 16)])
        pltpu.sync_copy(zbuf.at[pl.ds(0, 8)], acc.at[pl.ds(zb + rows - 8, 8)])

    zero_acc()
    plsc.subcore_barrier()

    def accumulate(msg_ref):
        def chunk(i, carry):
            off = pl.multiple_of(i * SK, 8)
            pltpu.sync_copy(msg_ref.at[pl.ds(ebase + off, SK)], mbuf)
            pltpu.sync_copy(mbuf, acc.at[ridx.at[i]], add=True)
            return carry
        lax.fori_loop(0, SC_CHUNKS, chunk, 0)

    @pl.when(cid == 0)
    def _():
        accumulate(msg_a)

    @pl.when(cid == 1)
    def _():
        accumulate(msg_b)

    plsc.subcore_barrier()

    @pl.when(cid == 0)
    def _():
        pltpu.sync_copy(acc.at[pl.ds(zb, rows)], sum_a_out.at[pl.ds(zb, rows)])

    @pl.when(cid == 1)
    def _():
        pltpu.sync_copy(acc.at[pl.ds(zb, rows)], sum_b_out.at[pl.ds(zb, rows)])

    zero_acc()
    plsc.subcore_barrier()

    def cchunk(i, carry):
        pltpu.sync_copy(ones, acc.at[ridx.at[i]], add=True)
        return carry

    @pl.when(cid == 0)
    def _():
        lax.fori_loop(0, NC0, cchunk, 0)

    @pl.when(cid == 1)
    def _():
        lax.fori_loop(NC0, SC_CHUNKS, cchunk, 0)

    plsc.subcore_barrier()

    @pl.when(cid == 0)
    def _():
        pltpu.sync_copy(acc.at[pl.ds(zb, rows)], cnt_a_out.at[pl.ds(zb, rows)])

    @pl.when(cid == 1)
    def _():
        pltpu.sync_copy(acc.at[pl.ds(zb, rows)], cnt_b_out.at[pl.ds(zb, rows)])


def _div_body(*refs):
    sa_refs = refs[:K]
    sb_refs = refs[K:2 * K]
    cn_refs = refs[2 * K:4 * K]
    o = refs[4 * K]
    sa = sa_refs[0][:]
    sb = sb_refs[0][:]
    for r in sa_refs[1:]:
        sa = sa + r[:]
    for r in sb_refs[1:]:
        sb = sb + r[:]
    cnt = cn_refs[0][:, 0:1]
    for r in cn_refs[1:]:
        cnt = cnt + r[:, 0:1]
    o[:] = jnp.concatenate([sa, sb], axis=1) / jnp.maximum(cnt, 1.0)


def _div_call(sum_as, sum_bs, cnts):
    BR = 128
    block = pl.BlockSpec((BR, 128), lambda i: (i, 0))
    n_in = 4 * K
    return pl.pallas_call(
        _div_body,
        grid=(NP2 // BR,),
        in_specs=[block] * n_in,
        out_specs=pl.BlockSpec((BR, DF), lambda i: (i, 0)),
        out_shape=jax.ShapeDtypeStruct((NP2, DF), jnp.float32),
    )(*sum_as, *sum_bs, *cnts)


def kernel(nodes, senders, receivers, edge_attr, globals, batch,
           eW1, eb1, eW2, eb2, nW1, nb1, nW2, nb2, gW1, gb1, gW2, gb2):
    nodes_p = jnp.pad(nodes, ((0, NP - N), (0, 0)))
    wsr = eW1[:2 * DF].astype(jnp.bfloat16)
    w1e = eW1[2 * DF:].astype(jnp.bfloat16)
    nw1s = nW1[:DF].astype(jnp.bfloat16)
    nw1e = nW1[DF:].astype(jnp.bfloat16)
    eW2b = eW2.astype(jnp.bfloat16)
    nW2b = nW2.astype(jnp.bfloat16)

    senders4 = senders.astype(jnp.int32).reshape(K, NW, GC, GK)
    receivers4 = receivers.astype(jnp.int32).reshape(K, NW, GC, GK)
    receivers4s = receivers.astype(jnp.int32).reshape(K, 16, SC_CHUNKS, SK)

    node_pack, g_out = _pre_call(
        nodes_p, globals.reshape(1, DG),
        gW1, gb1.reshape(1, DG), gW2, gb2.reshape(1, DG))

    eb1r = eb1.reshape(1, DH)
    eb2r = eb2.reshape(1, DEO)
    nb1r = nb1.reshape(1, DH)
    nb2r = nb2.reshape(1, DF)

    ne_parts, sum_as, sum_bs, cnts = [], [], [], []
    for k in range(K):
        gs, gr = _gather_kernel(node_pack, senders4[k], receivers4[k])
        ne_k, ma_k, mb_k = _edge_call(
            gs, gr, edge_attr[k * EC:(k + 1) * EC],
            wsr, w1e, eb1r, eW2b, eb2r, nw1s, nw1e, nb1r, nW2b, nb2r)
        sa_k, sb_k, ca_k, cb_k = _scatter_kernel(ma_k, mb_k, receivers4s[k])
        ne_parts.append(ne_k)
        sum_as.append(sa_k)
        sum_bs.append(sb_k)
        cnts.append(ca_k)
        cnts.append(cb_k)

    new_nodes = _div_call(sum_as, sum_bs, cnts)[:N]
    new_edge = jnp.concatenate(ne_parts, axis=0)
    return (new_nodes, new_edge, g_out.reshape(DG))


# speedup vs baseline: 2.1532x; 1.0149x over previous
import functools

import jax
import jax.numpy as jnp
from jax import lax
from jax.experimental import pallas as pl
from jax.experimental.pallas import tpu as pltpu
from jax.experimental.pallas import tpu_sc as plsc

N = 10000
E = 160000
DF = 256
DE = 16
DEO = 64
DH = 512
DG = 64

NP = 10240
NP2 = 10112
K = 5
EC = E // K
NW = 32
EPW = EC // NW
GK = 40
GC = EPW // GK
EPS = EC // 16
SK = 80
SC_CHUNKS = EPS // SK
NC0 = 13

_mesh = plsc.VectorSubcoreMesh(core_axis_name="c", subcore_axis_name="s")

_HI = -65536


def _pack(lo_f32, hi_f32):
    lo = lo_f32.astype(jnp.bfloat16).astype(jnp.float32)
    hi = hi_f32.astype(jnp.bfloat16).astype(jnp.float32)
    lo_b = lax.bitcast_convert_type(lo, jnp.int32)
    hi_b = lax.bitcast_convert_type(hi, jnp.int32)
    return (hi_b & _HI) | lax.shift_right_logical(lo_b, 16)


def _unpack_lo(w):
    return lax.bitcast_convert_type(lax.shift_left(w, 16), jnp.float32)


def _unpack_hi(w):
    return lax.bitcast_convert_type(w & _HI, jnp.float32)


def _pre_body(x, g, gw1, gb1, gw2, gb2, pack_out, g_out):
    xb = x[:]
    pack_out[:] = _pack(xb[:, :128], xb[:, 128:])

    @pl.when(pl.program_id(0) == 0)
    def _():
        h = jnp.maximum(
            jnp.dot(g[:], gw1[:], preferred_element_type=jnp.float32) + gb1[:], 0.0)
        g_out[:] = jnp.dot(h, gw2[:], preferred_element_type=jnp.float32) + gb2[:]


def _pre_call(nodes_p, g, gw1, gb1, gw2, gb2):
    full = lambda shape: pl.BlockSpec(shape, lambda i: (0, 0))
    return pl.pallas_call(
        _pre_body,
        grid=(NP // 256,),
        in_specs=[
            pl.BlockSpec((256, DF), lambda i: (i, 0)),
            full((1, DG)),
            full((DG, DG)),
            full((1, DG)),
            full((DG, DG)),
            full((1, DG)),
        ],
        out_specs=[
            pl.BlockSpec((256, 128), lambda i: (i, 0)),
            full((1, DG)),
        ],
        out_shape=[
            jax.ShapeDtypeStruct((NP, 128), jnp.int32),
            jax.ShapeDtypeStruct((1, DG), jnp.float32),
        ],
    )(nodes_p, g, gw1, gb1, gw2, gb2)


@functools.partial(
    pl.kernel,
    out_type=[
        jax.ShapeDtypeStruct((EC, 128), jnp.int32),
        jax.ShapeDtypeStruct((EC, 128), jnp.int32),
    ],
    mesh=_mesh,
    scratch_types=[
        pltpu.VMEM((GC, GK), jnp.int32),
        pltpu.VMEM((GC, GK), jnp.int32),
        pltpu.VMEM((2, GK, 128), jnp.int32),
        pltpu.VMEM((2, GK, 128), jnp.int32),
        pltpu.SemaphoreType.DMA,
        pltpu.SemaphoreType.DMA,
        pltpu.SemaphoreType.DMA,
        pltpu.SemaphoreType.DMA,
    ],
)
def _gather_kernel(tab, senders3, receivers3,
                   gs_out, gr_out,
                   sidx, ridx, sbuf, rbuf, sem_s0, sem_s1, sem_r0, sem_r1):
    cid = lax.axis_index("c")
    sid = lax.axis_index("s")
    wid = sid * 2 + cid
    base = pl.multiple_of(wid * EPW, 8)
    pltpu.sync_copy(senders3.at[wid], sidx)
    pltpu.sync_copy(receivers3.at[wid], ridx)

    sem_s = (sem_s0, sem_s1)
    sem_r = (sem_r0, sem_r1)

    def start(i):
        p = i % 2
        cs = pltpu.async_copy(tab.at[sidx.at[i]], sbuf.at[p], sem_s[p])
        cr = pltpu.async_copy(tab.at[ridx.at[i]], rbuf.at[p], sem_r[p])
        return cs, cr

    inflight = start(0)
    for i in range(GC):
        p = i % 2
        cs, cr = inflight
        if i + 1 < GC:
            nxt = start(i + 1)
        cs.wait()
        cr.wait()
        if i + 1 < GC:
            inflight = nxt
        off = pl.multiple_of(i * GK, 8)
        pltpu.sync_copy(sbuf.at[p], gs_out.at[pl.ds(base + off, GK)])
        pltpu.sync_copy(rbuf.at[p], gr_out.at[pl.ds(base + off, GK)])


def _edge_body(gs, gr, ea, wsr, w1e, b1, w2, b2, nw1s, nw1e, nb1, nw2, nb2,
               ne_out, ma_out, mb_out):
    ws = gs[:]
    s_b = jnp.concatenate(
        [_unpack_lo(ws), _unpack_hi(ws)], axis=1).astype(jnp.bfloat16)
    wr = gr[:]
    r_b = jnp.concatenate(
        [_unpack_lo(wr), _unpack_hi(wr)], axis=1).astype(jnp.bfloat16)
    x = jnp.concatenate([s_b, r_b], axis=1)
    h1 = jnp.maximum(
        jnp.dot(x, wsr[:], preferred_element_type=jnp.float32)
        + jnp.dot(ea[:].astype(jnp.bfloat16), w1e[:],
                  preferred_element_type=jnp.float32)
        + b1[:], 0.0)
    ne = jnp.dot(h1.astype(jnp.bfloat16), w2[:],
                 preferred_element_type=jnp.float32) + b2[:]
    ne_out[:] = ne
    h2 = jnp.maximum(
        jnp.dot(s_b, nw1s[:], preferred_element_type=jnp.float32)
        + jnp.dot(ne.astype(jnp.bfloat16), nw1e[:],
                  preferred_element_type=jnp.float32)
        + nb1[:], 0.0)
    m = jnp.dot(h2.astype(jnp.bfloat16), nw2[:],
                preferred_element_type=jnp.float32) + nb2[:]
    ma_out[:] = m[:, :128]
    mb_out[:] = m[:, 128:]


def _edge_call(gs, gr, ea, wsr, w1e, b1, w2, b2, nw1s, nw1e, nb1, nw2, nb2):
    BE = 256
    full = lambda shape: pl.BlockSpec(shape, lambda i: (0, 0))
    return pl.pallas_call(
        _edge_body,
        grid=(EC // BE,),
        in_specs=[
            pl.BlockSpec((BE, 128), lambda i: (i, 0)),
            pl.BlockSpec((BE, 128), lambda i: (i, 0)),
            pl.BlockSpec((BE, DE), lambda i: (i, 0)),
            full((2 * DF, DH)),
            full((DE, DH)),
            full((1, DH)),
            full((DH, DEO)),
            full((1, DEO)),
            full((DF, DH)),
            full((DEO, DH)),
            full((1, DH)),
            full((DH, DF)),
            full((1, DF)),
        ],
        out_specs=[
            pl.BlockSpec((BE, DEO), lambda i: (i, 0)),
            pl.BlockSpec((BE, 128), lambda i: (i, 0)),
            pl.BlockSpec((BE, 128), lambda i: (i, 0)),
        ],
        out_shape=[
            jax.ShapeDtypeStruct((EC, DEO), jnp.float32),
            jax.ShapeDtypeStruct((EC, 128), jnp.float32),
            jax.ShapeDtypeStruct((EC, 128), jnp.float32),
        ],
    )(gs, gr, ea, wsr, w1e, b1, w2, b2, nw1s, nw1e, nb1, nw2, nb2)


@functools.partial(
    pl.kernel,
    out_type=[
        jax.ShapeDtypeStruct((NP2, 128), jnp.float32),
        jax.ShapeDtypeStruct((NP2, 128), jnp.float32),
        jax.ShapeDtypeStruct((NP2, 128), jnp.float32),
        jax.ShapeDtypeStruct((NP2, 128), jnp.float32),
    ],
    mesh=_mesh,
    scratch_types=[
        pltpu.VMEM((SC_CHUNKS, SK), jnp.int32),
        pltpu.VMEM((2, SK, 128), jnp.float32),
        pltpu.VMEM((16, 128), jnp.float32),
        pltpu.VMEM((SK, 128), jnp.float32),
        pltpu.VMEM_SHARED((NP2, 128), jnp.float32),
        pltpu.SemaphoreType.DMA,
        pltpu.SemaphoreType.DMA,
    ],
)
def _scatter_kernel(msg_a, msg_b, receivers3,
                    sum_a_out, sum_b_out, cnt_a_out, cnt_b_out,
                    ridx, mbuf, zbuf, ones, acc, sem_m0, sem_m1):
    cid = lax.axis_index("c")
    sid = lax.axis_index("s")
    ebase = pl.multiple_of(sid * EPS, 8)
    pltpu.sync_copy(receivers3.at[sid], ridx)

    zero = jnp.zeros((16,), jnp.float32)
    one = jnp.ones((16,), jnp.float32)
    for r in range(16):
        for q in range(128 // 16):
            zbuf[r, pl.ds(q * 16, 16)] = zero
    for r in range(SK):
        for q in range(128 // 16):
            ones[r, pl.ds(q * 16, 16)] = one

    rows = NP2 // 16
    zb = sid * rows

    def zero_acc():
        for j in range(rows // 16):
            pltpu.sync_copy(zbuf, acc.at[pl.ds(zb + j * 16, 16)])
        pltpu.sync_copy(zbuf.at[pl.ds(0, 8)], acc.at[pl.ds(zb + rows - 8, 8)])

    zero_acc()
    plsc.subcore_barrier()

    sem_m = (sem_m0, sem_m1)

    def accumulate(msg_ref):
        def start(i):
            off = pl.multiple_of(i * SK, 8)
            p = i % 2
            return pltpu.async_copy(
                msg_ref.at[pl.ds(ebase + off, SK)], mbuf.at[p], sem_m[p])

        inflight = start(0)
        for i in range(SC_CHUNKS):
            cp = inflight
            if i + 1 < SC_CHUNKS:
                nxt = start(i + 1)
            cp.wait()
            if i + 1 < SC_CHUNKS:
                inflight = nxt
            pltpu.sync_copy(mbuf.at[i % 2], acc.at[ridx.at[i]], add=True)

    @pl.when(cid == 0)
    def _():
        accumulate(msg_a)

    @pl.when(cid == 1)
    def _():
        accumulate(msg_b)

    plsc.subcore_barrier()

    @pl.when(cid == 0)
    def _():
        pltpu.sync_copy(acc.at[pl.ds(zb, rows)], sum_a_out.at[pl.ds(zb, rows)])

    @pl.when(cid == 1)
    def _():
        pltpu.sync_copy(acc.at[pl.ds(zb, rows)], sum_b_out.at[pl.ds(zb, rows)])

    zero_acc()
    plsc.subcore_barrier()

    def cchunk(i, carry):
        pltpu.sync_copy(ones, acc.at[ridx.at[i]], add=True)
        return carry

    @pl.when(cid == 0)
    def _():
        lax.fori_loop(0, NC0, cchunk, 0)

    @pl.when(cid == 1)
    def _():
        lax.fori_loop(NC0, SC_CHUNKS, cchunk, 0)

    plsc.subcore_barrier()

    @pl.when(cid == 0)
    def _():
        pltpu.sync_copy(acc.at[pl.ds(zb, rows)], cnt_a_out.at[pl.ds(zb, rows)])

    @pl.when(cid == 1)
    def _():
        pltpu.sync_copy(acc.at[pl.ds(zb, rows)], cnt_b_out.at[pl.ds(zb, rows)])


def _div_body(*refs):
    sa_refs = refs[:K]
    sb_refs = refs[K:2 * K]
    cn_refs = refs[2 * K:4 * K]
    o = refs[4 * K]
    sa = sa_refs[0][:]
    sb = sb_refs[0][:]
    for r in sa_refs[1:]:
        sa = sa + r[:]
    for r in sb_refs[1:]:
        sb = sb + r[:]
    cnt = cn_refs[0][:, 0:1]
    for r in cn_refs[1:]:
        cnt = cnt + r[:, 0:1]
    o[:] = jnp.concatenate([sa, sb], axis=1) / jnp.maximum(cnt, 1.0)


def _div_call(sum_as, sum_bs, cnts):
    BR = 128
    block = pl.BlockSpec((BR, 128), lambda i: (i, 0))
    n_in = 4 * K
    return pl.pallas_call(
        _div_body,
        grid=(NP2 // BR,),
        in_specs=[block] * n_in,
        out_specs=pl.BlockSpec((BR, DF), lambda i: (i, 0)),
        out_shape=jax.ShapeDtypeStruct((NP2, DF), jnp.float32),
    )(*sum_as, *sum_bs, *cnts)


def kernel(nodes, senders, receivers, edge_attr, globals, batch,
           eW1, eb1, eW2, eb2, nW1, nb1, nW2, nb2, gW1, gb1, gW2, gb2):
    nodes_p = jnp.pad(nodes, ((0, NP - N), (0, 0)))
    wsr = eW1[:2 * DF].astype(jnp.bfloat16)
    w1e = eW1[2 * DF:].astype(jnp.bfloat16)
    nw1s = nW1[:DF].astype(jnp.bfloat16)
    nw1e = nW1[DF:].astype(jnp.bfloat16)
    eW2b = eW2.astype(jnp.bfloat16)
    nW2b = nW2.astype(jnp.bfloat16)

    senders4 = senders.astype(jnp.int32).reshape(K, NW, GC, GK)
    receivers4 = receivers.astype(jnp.int32).reshape(K, NW, GC, GK)
    receivers4s = receivers.astype(jnp.int32).reshape(K, 16, SC_CHUNKS, SK)

    node_pack, g_out = _pre_call(
        nodes_p, globals.reshape(1, DG),
        gW1, gb1.reshape(1, DG), gW2, gb2.reshape(1, DG))

    eb1r = eb1.reshape(1, DH)
    eb2r = eb2.reshape(1, DEO)
    nb1r = nb1.reshape(1, DH)
    nb2r = nb2.reshape(1, DF)

    ne_parts, sum_as, sum_bs, cnts = [], [], [], []
    for k in range(K):
        gs, gr = _gather_kernel(node_pack, senders4[k], receivers4[k])
        ne_k, ma_k, mb_k = _edge_call(
            gs, gr, edge_attr[k * EC:(k + 1) * EC],
            wsr, w1e, eb1r, eW2b, eb2r, nw1s, nw1e, nb1r, nW2b, nb2r)
        sa_k, sb_k, ca_k, cb_k = _scatter_kernel(ma_k, mb_k, receivers4s[k])
        ne_parts.append(ne_k)
        sum_as.append(sa_k)
        sum_bs.append(sb_k)
        cnts.append(ca_k)
        cnts.append(cb_k)

    new_nodes = _div_call(sum_as, sum_bs, cnts)[:N]
    new_edge = jnp.concatenate(ne_parts, axis=0)
    return (new_nodes, new_edge, g_out.reshape(DG))


# speedup vs baseline: 2.1621x; 1.0041x over previous
import functools

import jax
import jax.numpy as jnp
from jax import lax
from jax.experimental import pallas as pl
from jax.experimental.pallas import tpu as pltpu
from jax.experimental.pallas import tpu_sc as plsc

N = 10000
E = 160000
DF = 256
DE = 16
DEO = 64
DH = 512
DG = 64

NP = 10240
NP2 = 10112
K = 5
EC = E // K
NW = 32
EPW = EC // NW
GK = 40
GC = EPW // GK
EPS = EC // 16
SK = 80
SC_CHUNKS = EPS // SK
NC0 = 13

_mesh = plsc.VectorSubcoreMesh(core_axis_name="c", subcore_axis_name="s")

_HI = -65536


def _pack(lo_f32, hi_f32):
    lo = lo_f32.astype(jnp.bfloat16).astype(jnp.float32)
    hi = hi_f32.astype(jnp.bfloat16).astype(jnp.float32)
    lo_b = lax.bitcast_convert_type(lo, jnp.int32)
    hi_b = lax.bitcast_convert_type(hi, jnp.int32)
    return (hi_b & _HI) | lax.shift_right_logical(lo_b, 16)


def _unpack_lo(w):
    return lax.bitcast_convert_type(lax.shift_left(w, 16), jnp.float32)


def _unpack_hi(w):
    return lax.bitcast_convert_type(w & _HI, jnp.float32)


def _pre_body(x, g, gw1, gb1, gw2, gb2, pack_out, g_out):
    xb = x[:]
    pack_out[:] = _pack(xb[:, :128], xb[:, 128:])

    @pl.when(pl.program_id(0) == 0)
    def _():
        h = jnp.maximum(
            jnp.dot(g[:], gw1[:], preferred_element_type=jnp.float32) + gb1[:], 0.0)
        g_out[:] = jnp.dot(h, gw2[:], preferred_element_type=jnp.float32) + gb2[:]


def _pre_call(nodes_p, g, gw1, gb1, gw2, gb2):
    full = lambda shape: pl.BlockSpec(shape, lambda i: (0, 0))
    return pl.pallas_call(
        _pre_body,
        grid=(NP // 256,),
        in_specs=[
            pl.BlockSpec((256, DF), lambda i: (i, 0)),
            full((1, DG)),
            full((DG, DG)),
            full((1, DG)),
            full((DG, DG)),
            full((1, DG)),
        ],
        out_specs=[
            pl.BlockSpec((256, 128), lambda i: (i, 0)),
            full((1, DG)),
        ],
        out_shape=[
            jax.ShapeDtypeStruct((NP, 128), jnp.int32),
            jax.ShapeDtypeStruct((1, DG), jnp.float32),
        ],
    )(nodes_p, g, gw1, gb1, gw2, gb2)


@functools.partial(
    pl.kernel,
    out_type=[
        jax.ShapeDtypeStruct((EC, 128), jnp.int32),
        jax.ShapeDtypeStruct((EC, 128), jnp.int32),
    ],
    mesh=_mesh,
    scratch_types=[
        pltpu.VMEM((GC, GK), jnp.int32),
        pltpu.VMEM((GC, GK), jnp.int32),
        pltpu.VMEM((2, GK, 128), jnp.int32),
        pltpu.VMEM((2, GK, 128), jnp.int32),
        pltpu.SemaphoreType.DMA,
        pltpu.SemaphoreType.DMA,
        pltpu.SemaphoreType.DMA,
        pltpu.SemaphoreType.DMA,
    ],
)
def _gather_kernel(tab, senders3, receivers3,
                   gs_out, gr_out,
                   sidx, ridx, sbuf, rbuf, sem_s0, sem_s1, sem_r0, sem_r1):
    cid = lax.axis_index("c")
    sid = lax.axis_index("s")
    wid = sid * 2 + cid
    base = pl.multiple_of(wid * EPW, 8)
    pltpu.sync_copy(senders3.at[wid], sidx)
    pltpu.sync_copy(receivers3.at[wid], ridx)

    sem_s = (sem_s0, sem_s1)
    sem_r = (sem_r0, sem_r1)

    def start(i):
        p = i % 2
        cs = pltpu.async_copy(tab.at[sidx.at[i]], sbuf.at[p], sem_s[p])
        cr = pltpu.async_copy(tab.at[ridx.at[i]], rbuf.at[p], sem_r[p])
        return cs, cr

    inflight = start(0)
    for i in range(GC):
        p = i % 2
        cs, cr = inflight
        if i + 1 < GC:
            nxt = start(i + 1)
        cs.wait()
        cr.wait()
        if i + 1 < GC:
            inflight = nxt
        off = pl.multiple_of(i * GK, 8)
        pltpu.sync_copy(sbuf.at[p], gs_out.at[pl.ds(base + off, GK)])
        pltpu.sync_copy(rbuf.at[p], gr_out.at[pl.ds(base + off, GK)])


def _edge_body(gs, gr, ea, wsr, w1e, b1, w2, b2, nw1s, nw1e, nb1, nw2, nb2,
               ne_out, ma_out, mb_out):
    ws = gs[:]
    s_b = jnp.concatenate(
        [_unpack_lo(ws), _unpack_hi(ws)], axis=1).astype(jnp.bfloat16)
    wr = gr[:]
    r_b = jnp.concatenate(
        [_unpack_lo(wr), _unpack_hi(wr)], axis=1).astype(jnp.bfloat16)
    x = jnp.concatenate([s_b, r_b], axis=1)
    h1 = jnp.maximum(
        jnp.dot(x, wsr[:], preferred_element_type=jnp.float32)
        + jnp.dot(ea[:].astype(jnp.bfloat16), w1e[:],
                  preferred_element_type=jnp.float32)
        + b1[:], 0.0)
    ne = jnp.dot(h1.astype(jnp.bfloat16), w2[:],
                 preferred_element_type=jnp.float32) + b2[:]
    ne_out[:] = ne
    h2 = jnp.maximum(
        jnp.dot(s_b, nw1s[:], preferred_element_type=jnp.float32)
        + jnp.dot(ne.astype(jnp.bfloat16), nw1e[:],
                  preferred_element_type=jnp.float32)
        + nb1[:], 0.0)
    m = jnp.dot(h2.astype(jnp.bfloat16), nw2[:],
                preferred_element_type=jnp.float32) + nb2[:]
    ma_out[:] = m[:, :128]
    mb_out[:] = m[:, 128:]


def _edge_call(gs, gr, ea, wsr, w1e, b1, w2, b2, nw1s, nw1e, nb1, nw2, nb2):
    BE = 256
    full = lambda shape: pl.BlockSpec(shape, lambda i: (0, 0))
    return pl.pallas_call(
        _edge_body,
        grid=(EC // BE,),
        in_specs=[
            pl.BlockSpec((BE, 128), lambda i: (i, 0)),
            pl.BlockSpec((BE, 128), lambda i: (i, 0)),
            pl.BlockSpec((BE, DE), lambda i: (i, 0)),
            full((2 * DF, DH)),
            full((DE, DH)),
            full((1, DH)),
            full((DH, DEO)),
            full((1, DEO)),
            full((DF, DH)),
            full((DEO, DH)),
            full((1, DH)),
            full((DH, DF)),
            full((1, DF)),
        ],
        out_specs=[
            pl.BlockSpec((BE, DEO), lambda i: (i, 0)),
            pl.BlockSpec((BE, 128), lambda i: (i, 0)),
            pl.BlockSpec((BE, 128), lambda i: (i, 0)),
        ],
        out_shape=[
            jax.ShapeDtypeStruct((EC, DEO), jnp.float32),
            jax.ShapeDtypeStruct((EC, 128), jnp.float32),
            jax.ShapeDtypeStruct((EC, 128), jnp.float32),
        ],
    )(gs, gr, ea, wsr, w1e, b1, w2, b2, nw1s, nw1e, nb1, nw2, nb2)


@functools.partial(
    pl.kernel,
    out_type=[
        jax.ShapeDtypeStruct((NP2, 128), jnp.float32),
        jax.ShapeDtypeStruct((NP2, 128), jnp.float32),
        jax.ShapeDtypeStruct((NP2, 128), jnp.float32),
        jax.ShapeDtypeStruct((NP2, 128), jnp.float32),
    ],
    mesh=_mesh,
    scratch_types=[
        pltpu.VMEM((SC_CHUNKS, SK), jnp.int32),
        pltpu.VMEM((2, SK, 128), jnp.float32),
        pltpu.VMEM((16, 128), jnp.float32),
        pltpu.VMEM((SK, 128), jnp.float32),
        pltpu.VMEM_SHARED((NP2, 128), jnp.float32),
        pltpu.SemaphoreType.DMA,
        pltpu.SemaphoreType.DMA,
    ],
)
def _scatter_kernel(msg_a, msg_b, receivers3,
                    sum_a_out, sum_b_out, cnt_a_out, cnt_b_out,
                    ridx, mbuf, zbuf, ones, acc, sem_m0, sem_m1):
    cid = lax.axis_index("c")
    sid = lax.axis_index("s")
    ebase = pl.multiple_of(sid * EPS, 8)
    pltpu.sync_copy(receivers3.at[sid], ridx)

    zero = jnp.zeros((16,), jnp.float32)
    one = jnp.ones((16,), jnp.float32)
    for r in range(16):
        for q in range(128 // 16):
            zbuf[r, pl.ds(q * 16, 16)] = zero
    for r in range(SK):
        for q in range(128 // 16):
            ones[r, pl.ds(q * 16, 16)] = one

    rows = NP2 // 16
    zb = sid * rows

    def zero_acc():
        for j in range(rows // 16):
            pltpu.sync_copy(zbuf, acc.at[pl.ds(zb + j * 16, 16)])
        pltpu.sync_copy(zbuf.at[pl.ds(0, 8)], acc.at[pl.ds(zb + rows - 8, 8)])

    zero_acc()
    plsc.subcore_barrier()

    sem_m = (sem_m0, sem_m1)

    def accumulate(msg_ref):
        def start(i):
            off = pl.multiple_of(i * SK, 8)
            p = i % 2
            return pltpu.async_copy(
                msg_ref.at[pl.ds(ebase + off, SK)], mbuf.at[p], sem_m[p])

        inflight = start(0)
        for i in range(SC_CHUNKS):
            cp = inflight
            if i + 1 < SC_CHUNKS:
                nxt = start(i + 1)
            cp.wait()
            if i + 1 < SC_CHUNKS:
                inflight = nxt
            pltpu.sync_copy(mbuf.at[i % 2], acc.at[ridx.at[i]], add=True)

    @pl.when(cid == 0)
    def _():
        accumulate(msg_a)

    @pl.when(cid == 1)
    def _():
        accumulate(msg_b)

    plsc.subcore_barrier()

    @pl.when(cid == 0)
    def _():
        pltpu.sync_copy(acc.at[pl.ds(zb, rows)], sum_a_out.at[pl.ds(zb, rows)])

    @pl.when(cid == 1)
    def _():
        pltpu.sync_copy(acc.at[pl.ds(zb, rows)], sum_b_out.at[pl.ds(zb, rows)])

    zero_acc()
    plsc.subcore_barrier()

    def cchunk(i, carry):
        pltpu.sync_copy(ones, acc.at[ridx.at[i]], add=True)
        return carry

    @pl.when(cid == 0)
    def _():
        lax.fori_loop(0, NC0, cchunk, 0)

    @pl.when(cid == 1)
    def _():
        lax.fori_loop(NC0, SC_CHUNKS, cchunk, 0)

    plsc.subcore_barrier()

    @pl.when(cid == 0)
    def _():
        pltpu.sync_copy(acc.at[pl.ds(zb, rows)], cnt_a_out.at[pl.ds(zb, rows)])

    @pl.when(cid == 1)
    def _():
        pltpu.sync_copy(acc.at[pl.ds(zb, rows)], cnt_b_out.at[pl.ds(zb, rows)])


def _div_body(*refs):
    sa_refs = refs[:K]
    sb_refs = refs[K:2 * K]
    cn_refs = refs[2 * K:4 * K]
    o = refs[4 * K]
    sa = sa_refs[0][:]
    sb = sb_refs[0][:]
    for r in sa_refs[1:]:
        sa = sa + r[:]
    for r in sb_refs[1:]:
        sb = sb + r[:]
    cnt = cn_refs[0][:, 0:1]
    for r in cn_refs[1:]:
        cnt = cnt + r[:, 0:1]
    o[:] = jnp.concatenate([sa, sb], axis=1) / jnp.maximum(cnt, 1.0)


def _div_call(sum_as, sum_bs, cnts):
    BR = 128
    block = pl.BlockSpec((BR, 128), lambda i: (i, 0))
    n_in = 4 * K
    return pl.pallas_call(
        _div_body,
        grid=(NP2 // BR,),
        in_specs=[block] * n_in,
        out_specs=pl.BlockSpec((BR, DF), lambda i: (i, 0)),
        out_shape=jax.ShapeDtypeStruct((NP2, DF), jnp.float32),
    )(*sum_as, *sum_bs, *cnts)


def kernel(nodes, senders, receivers, edge_attr, globals, batch,
           eW1, eb1, eW2, eb2, nW1, nb1, nW2, nb2, gW1, gb1, gW2, gb2):
    nodes_p = jnp.pad(nodes, ((0, NP - N), (0, 0)))
    wsr = eW1[:2 * DF].astype(jnp.bfloat16)
    w1e = eW1[2 * DF:].astype(jnp.bfloat16)
    nw1s = nW1[:DF].astype(jnp.bfloat16)
    nw1e = nW1[DF:].astype(jnp.bfloat16)
    eW2b = eW2.astype(jnp.bfloat16)
    nW2b = nW2.astype(jnp.bfloat16)

    senders4 = senders.astype(jnp.int32).reshape(K, NW, GC, GK)
    receivers4 = receivers.astype(jnp.int32).reshape(K, NW, GC, GK)
    receivers4s = receivers.astype(jnp.int32).reshape(K, 16, SC_CHUNKS, SK)

    node_pack, g_out = _pre_call(
        nodes_p, globals.reshape(1, DG),
        gW1, gb1.reshape(1, DG), gW2, gb2.reshape(1, DG))

    eb1r = eb1.reshape(1, DH)
    eb2r = eb2.reshape(1, DEO)
    nb1r = nb1.reshape(1, DH)
    nb2r = nb2.reshape(1, DF)

    ne_parts, sum_as, sum_bs, cnts = [], [], [], []
    g = [None] * K
    g[0] = _gather_kernel(node_pack, senders4[0], receivers4[0])
    for k in range(K):
        if k + 1 < K:
            g[k + 1] = _gather_kernel(
                node_pack, senders4[k + 1], receivers4[k + 1])
        gs, gr = g[k]
        ne_k, ma_k, mb_k = _edge_call(
            gs, gr, edge_attr[k * EC:(k + 1) * EC],
            wsr, w1e, eb1r, eW2b, eb2r, nw1s, nw1e, nb1r, nW2b, nb2r)
        sa_k, sb_k, ca_k, cb_k = _scatter_kernel(ma_k, mb_k, receivers4s[k])
        ne_parts.append(ne_k)
        sum_as.append(sa_k)
        sum_bs.append(sb_k)
        cnts.append(ca_k)
        cnts.append(cb_k)

    new_nodes = _div_call(sum_as, sum_bs, cnts)[:N]
    new_edge = jnp.concatenate(ne_parts, axis=0)
    return (new_nodes, new_edge, g_out.reshape(DG))


# speedup vs baseline: 2.7927x; 1.2917x over previous
import functools

import jax
import jax.numpy as jnp
from jax import lax
from jax.experimental import pallas as pl
from jax.experimental.pallas import tpu as pltpu
from jax.experimental.pallas import tpu_sc as plsc

N = 10000
E = 160000
DF = 256
DE = 16
DEO = 64
DH = 512
DG = 64

NP = 10240
NP2 = 10112
K = 5
EC = E // K
NW = 32
EPW = EC // NW
GK = 40
GC = EPW // GK
EPS = EC // 16
SK = 80
SC_CHUNKS = EPS // SK
NC0 = 13

_mesh = plsc.VectorSubcoreMesh(core_axis_name="c", subcore_axis_name="s")

_HI = -65536


def _pack(lo_f32, hi_f32):
    lo = lo_f32.astype(jnp.bfloat16).astype(jnp.float32)
    hi = hi_f32.astype(jnp.bfloat16).astype(jnp.float32)
    lo_b = lax.bitcast_convert_type(lo, jnp.int32)
    hi_b = lax.bitcast_convert_type(hi, jnp.int32)
    return (hi_b & _HI) | lax.shift_right_logical(lo_b, 16)


def _unpack_lo(w):
    return lax.bitcast_convert_type(lax.shift_left(w, 16), jnp.float32)


def _unpack_hi(w):
    return lax.bitcast_convert_type(w & _HI, jnp.float32)


def _pre_body(x, g, gw1, gb1, gw2, gb2, pack_out, g_out):
    xb = x[:]
    pack_out[:] = _pack(xb[:, :128], xb[:, 128:])

    @pl.when(pl.program_id(0) == 0)
    def _():
        h = jnp.maximum(
            jnp.dot(g[:], gw1[:], preferred_element_type=jnp.float32) + gb1[:], 0.0)
        g_out[:] = jnp.dot(h, gw2[:], preferred_element_type=jnp.float32) + gb2[:]


def _pre_call(nodes_p, g, gw1, gb1, gw2, gb2):
    full = lambda shape: pl.BlockSpec(shape, lambda i: (0, 0))
    return pl.pallas_call(
        _pre_body,
        grid=(NP // 256,),
        in_specs=[
            pl.BlockSpec((256, DF), lambda i: (i, 0)),
            full((1, DG)),
            full((DG, DG)),
            full((1, DG)),
            full((DG, DG)),
            full((1, DG)),
        ],
        out_specs=[
            pl.BlockSpec((256, 128), lambda i: (i, 0)),
            full((1, DG)),
        ],
        out_shape=[
            jax.ShapeDtypeStruct((NP, 128), jnp.int32),
            jax.ShapeDtypeStruct((1, DG), jnp.float32),
        ],
    )(nodes_p, g, gw1, gb1, gw2, gb2)


@functools.partial(
    pl.kernel,
    out_type=[
        jax.ShapeDtypeStruct((EC, 128), jnp.int32),
        jax.ShapeDtypeStruct((EC, 128), jnp.int32),
    ],
    mesh=_mesh,
    scratch_types=[
        pltpu.VMEM((GC, GK), jnp.int32),
        pltpu.VMEM((GC, GK), jnp.int32),
        pltpu.VMEM((2, GK, 128), jnp.int32),
        pltpu.VMEM((2, GK, 128), jnp.int32),
        pltpu.SemaphoreType.DMA,
        pltpu.SemaphoreType.DMA,
        pltpu.SemaphoreType.DMA,
        pltpu.SemaphoreType.DMA,
    ],
)
def _gather_kernel(tab, senders3, receivers3,
                   gs_out, gr_out,
                   sidx, ridx, sbuf, rbuf, sem_s0, sem_s1, sem_r0, sem_r1):
    cid = lax.axis_index("c")
    sid = lax.axis_index("s")
    wid = sid * 2 + cid
    base = pl.multiple_of(wid * EPW, 8)
    pltpu.sync_copy(senders3.at[wid], sidx)
    pltpu.sync_copy(receivers3.at[wid], ridx)

    sem_s = (sem_s0, sem_s1)
    sem_r = (sem_r0, sem_r1)

    def start(i):
        p = i % 2
        cs = pltpu.async_copy(tab.at[sidx.at[i]], sbuf.at[p], sem_s[p])
        cr = pltpu.async_copy(tab.at[ridx.at[i]], rbuf.at[p], sem_r[p])
        return cs, cr

    inflight = start(0)
    for i in range(GC):
        p = i % 2
        cs, cr = inflight
        if i + 1 < GC:
            nxt = start(i + 1)
        cs.wait()
        cr.wait()
        if i + 1 < GC:
            inflight = nxt
        off = pl.multiple_of(i * GK, 8)
        pltpu.sync_copy(sbuf.at[p], gs_out.at[pl.ds(base + off, GK)])
        pltpu.sync_copy(rbuf.at[p], gr_out.at[pl.ds(base + off, GK)])


def _edge_body(gs, gr, ea, ws_p, wr_p, w1e, b1, w2, b2, nw1s_p, nw1e, nb1,
               nw2, nb2, ne_out, ma_out, mb_out):
    ws = gs[:]
    s_b = jnp.concatenate(
        [_unpack_lo(ws), _unpack_hi(ws)], axis=1).astype(jnp.bfloat16)
    wr = gr[:]
    r_b = jnp.concatenate(
        [_unpack_lo(wr), _unpack_hi(wr)], axis=1).astype(jnp.bfloat16)
    h1 = jnp.maximum(
        jnp.dot(s_b, ws_p[:], preferred_element_type=jnp.float32)
        + jnp.dot(r_b, wr_p[:], preferred_element_type=jnp.float32)
        + jnp.dot(ea[:].astype(jnp.bfloat16), w1e[:],
                  preferred_element_type=jnp.float32)
        + b1[:], 0.0)
    ne = jnp.dot(h1.astype(jnp.bfloat16), w2[:],
                 preferred_element_type=jnp.float32) + b2[:]
    ne_out[:] = ne
    h2 = jnp.maximum(
        jnp.dot(s_b, nw1s_p[:], preferred_element_type=jnp.float32)
        + jnp.dot(ne.astype(jnp.bfloat16), nw1e[:],
                  preferred_element_type=jnp.float32)
        + nb1[:], 0.0)
    m = jnp.dot(h2.astype(jnp.bfloat16), nw2[:],
                preferred_element_type=jnp.float32) + nb2[:]
    ma_out[:] = m[:, :128]
    mb_out[:] = m[:, 128:]


def _edge_call(gs, gr, ea, ws_p, wr_p, w1e, b1, w2, b2, nw1s_p, nw1e, nb1,
               nw2, nb2):
    BE = 640
    full = lambda shape: pl.BlockSpec(shape, lambda i: (0, 0))
    return pl.pallas_call(
        _edge_body,
        grid=(EC // BE,),
        in_specs=[
            pl.BlockSpec((BE, 128), lambda i: (i, 0)),
            pl.BlockSpec((BE, 128), lambda i: (i, 0)),
            pl.BlockSpec((BE, DE), lambda i: (i, 0)),
            full((DF, DH)),
            full((DF, DH)),
            full((DE, DH)),
            full((1, DH)),
            full((DH, DEO)),
            full((1, DEO)),
            full((DF, DH)),
            full((DEO, DH)),
            full((1, DH)),
            full((DH, DF)),
            full((1, DF)),
        ],
        out_specs=[
            pl.BlockSpec((BE, DEO), lambda i: (i, 0)),
            pl.BlockSpec((BE, 128), lambda i: (i, 0)),
            pl.BlockSpec((BE, 128), lambda i: (i, 0)),
        ],
        out_shape=[
            jax.ShapeDtypeStruct((EC, DEO), jnp.float32),
            jax.ShapeDtypeStruct((EC, 128), jnp.float32),
            jax.ShapeDtypeStruct((EC, 128), jnp.float32),
        ],
    )(gs, gr, ea, ws_p, wr_p, w1e, b1, w2, b2, nw1s_p, nw1e, nb1, nw2, nb2)


@functools.partial(
    pl.kernel,
    out_type=[
        jax.ShapeDtypeStruct((NP2, 128), jnp.float32),
        jax.ShapeDtypeStruct((NP2, 128), jnp.float32),
    ],
    mesh=_mesh,
    scratch_types=[
        pltpu.VMEM((SC_CHUNKS, SK), jnp.int32),
        pltpu.VMEM((2, SK, 128), jnp.float32),
        pltpu.VMEM((16, 128), jnp.float32),
        pltpu.VMEM_SHARED((NP2, 128), jnp.float32),
        pltpu.SemaphoreType.DMA,
        pltpu.SemaphoreType.DMA,
    ],
)
def _scatter_kernel(msg_a, msg_b, receivers3,
                    sum_a_out, sum_b_out,
                    ridx, mbuf, zbuf, acc, sem_m0, sem_m1):
    cid = lax.axis_index("c")
    sid = lax.axis_index("s")
    ebase = pl.multiple_of(sid * EPS, 8)
    pltpu.sync_copy(receivers3.at[sid], ridx)

    zero = jnp.zeros((16,), jnp.float32)
    for r in range(16):
        for q in range(128 // 16):
            zbuf[r, pl.ds(q * 16, 16)] = zero

    rows = NP2 // 16
    zb = sid * rows

    for j in range(rows // 16):
        pltpu.sync_copy(zbuf, acc.at[pl.ds(zb + j * 16, 16)])
    pltpu.sync_copy(zbuf.at[pl.ds(0, 8)], acc.at[pl.ds(zb + rows - 8, 8)])
    plsc.subcore_barrier()

    sem_m = (sem_m0, sem_m1)

    def accumulate(msg_ref):
        def start(i):
            off = pl.multiple_of(i * SK, 8)
            p = i % 2
            return pltpu.async_copy(
                msg_ref.at[pl.ds(ebase + off, SK)], mbuf.at[p], sem_m[p])

        inflight = start(0)
        for i in range(SC_CHUNKS):
            cp = inflight
            if i + 1 < SC_CHUNKS:
                nxt = start(i + 1)
            cp.wait()
            if i + 1 < SC_CHUNKS:
                inflight = nxt
            pltpu.sync_copy(mbuf.at[i % 2], acc.at[ridx.at[i]], add=True)

    @pl.when(cid == 0)
    def _():
        accumulate(msg_a)

    @pl.when(cid == 1)
    def _():
        accumulate(msg_b)

    plsc.subcore_barrier()

    @pl.when(cid == 0)
    def _():
        pltpu.sync_copy(acc.at[pl.ds(zb, rows)], sum_a_out.at[pl.ds(zb, rows)])

    @pl.when(cid == 1)
    def _():
        pltpu.sync_copy(acc.at[pl.ds(zb, rows)], sum_b_out.at[pl.ds(zb, rows)])


CK = 40
CCH = E // NW // CK


@functools.partial(
    pl.kernel,
    out_type=[
        jax.ShapeDtypeStruct((NP2, 128), jnp.float32),
        jax.ShapeDtypeStruct((NP2, 128), jnp.float32),
    ],
    mesh=_mesh,
    scratch_types=[
        pltpu.VMEM((CCH, CK), jnp.int32),
        pltpu.VMEM((16, 128), jnp.float32),
        pltpu.VMEM((CK, 128), jnp.float32),
        pltpu.VMEM_SHARED((NP2, 128), jnp.float32),
    ],
)
def _count_kernel(receivers3, cnt_a_out, cnt_b_out, ridx, zbuf, ones, acc):
    cid = lax.axis_index("c")
    sid = lax.axis_index("s")
    wid = sid * 2 + cid
    pltpu.sync_copy(receivers3.at[wid], ridx)

    zero = jnp.zeros((16,), jnp.float32)
    one = jnp.ones((16,), jnp.float32)
    for r in range(16):
        for q in range(128 // 16):
            zbuf[r, pl.ds(q * 16, 16)] = zero
    for r in range(CK):
        for q in range(128 // 16):
            ones[r, pl.ds(q * 16, 16)] = one

    rows = NP2 // 16
    zb = sid * rows
    for j in range(rows // 16):
        pltpu.sync_copy(zbuf, acc.at[pl.ds(zb + j * 16, 16)])
    pltpu.sync_copy(zbuf.at[pl.ds(0, 8)], acc.at[pl.ds(zb + rows - 8, 8)])
    plsc.subcore_barrier()

    def cchunk(i, carry):
        pltpu.sync_copy(ones, acc.at[ridx.at[i]], add=True)
        return carry

    lax.fori_loop(0, CCH, cchunk, 0)
    plsc.subcore_barrier()

    @pl.when(cid == 0)
    def _():
        pltpu.sync_copy(acc.at[pl.ds(zb, rows)], cnt_a_out.at[pl.ds(zb, rows)])

    @pl.when(cid == 1)
    def _():
        pltpu.sync_copy(acc.at[pl.ds(zb, rows)], cnt_b_out.at[pl.ds(zb, rows)])


def _div_body(*refs):
    sa_refs = refs[:K]
    sb_refs = refs[K:2 * K]
    cn_refs = refs[2 * K:2 * K + 2]
    o = refs[2 * K + 2]
    sa = sa_refs[0][:]
    sb = sb_refs[0][:]
    for r in sa_refs[1:]:
        sa = sa + r[:]
    for r in sb_refs[1:]:
        sb = sb + r[:]
    cnt = cn_refs[0][:, 0:1] + cn_refs[1][:, 0:1]
    o[:] = jnp.concatenate([sa, sb], axis=1) / jnp.maximum(cnt, 1.0)


def _div_call(sum_as, sum_bs, cnts):
    BR = 128
    block = pl.BlockSpec((BR, 128), lambda i: (i, 0))
    n_in = 2 * K + 2
    return pl.pallas_call(
        _div_body,
        grid=(NP2 // BR,),
        in_specs=[block] * n_in,
        out_specs=pl.BlockSpec((BR, DF), lambda i: (i, 0)),
        out_shape=jax.ShapeDtypeStruct((NP2, DF), jnp.float32),
    )(*sum_as, *sum_bs, *cnts)


def kernel(nodes, senders, receivers, edge_attr, globals, batch,
           eW1, eb1, eW2, eb2, nW1, nb1, nW2, nb2, gW1, gb1, gW2, gb2):
    nodes_p = jnp.pad(nodes, ((0, NP - N), (0, 0)))
    ws_p = eW1[:DF].astype(jnp.bfloat16)
    wr_p = eW1[DF:2 * DF].astype(jnp.bfloat16)
    w1e = eW1[2 * DF:].astype(jnp.bfloat16)
    nw1s_p = nW1[:DF].astype(jnp.bfloat16)
    nw1e = nW1[DF:].astype(jnp.bfloat16)
    eW2b = eW2.astype(jnp.bfloat16)
    nW2b = nW2.astype(jnp.bfloat16)

    senders4 = senders.astype(jnp.int32).reshape(K, NW, GC, GK)
    receivers4 = receivers.astype(jnp.int32).reshape(K, NW, GC, GK)
    receivers4s = receivers.astype(jnp.int32).reshape(K, 16, SC_CHUNKS, SK)
    receivers3c = receivers.astype(jnp.int32).reshape(NW, CCH, CK)

    node_pack, g_out = _pre_call(
        nodes_p, globals.reshape(1, DG),
        gW1, gb1.reshape(1, DG), gW2, gb2.reshape(1, DG))

    eb1r = eb1.reshape(1, DH)
    eb2r = eb2.reshape(1, DEO)
    nb1r = nb1.reshape(1, DH)
    nb2r = nb2.reshape(1, DF)

    ne_parts, sum_as, sum_bs = [], [], []
    g = [None] * K
    g[0] = _gather_kernel(node_pack, senders4[0], receivers4[0])
    cnt_a, cnt_b = _count_kernel(receivers3c)
    for k in range(K):
        if k + 1 < K:
            g[k + 1] = _gather_kernel(
                node_pack, senders4[k + 1], receivers4[k + 1])
        gs, gr = g[k]
        ne_k, ma_k, mb_k = _edge_call(
            gs, gr, edge_attr[k * EC:(k + 1) * EC],
            ws_p, wr_p, w1e, eb1r, eW2b, eb2r, nw1s_p, nw1e, nb1r,
            nW2b, nb2r)
        sa_k, sb_k = _scatter_kernel(ma_k, mb_k, receivers4s[k])
        ne_parts.append(ne_k)
        sum_as.append(sa_k)
        sum_bs.append(sb_k)

    new_nodes = _div_call(sum_as, sum_bs, [cnt_a, cnt_b])[:N]
    new_edge = jnp.concatenate(ne_parts, axis=0)
    return (new_nodes, new_edge, g_out.reshape(DG))


# speedup vs baseline: 3.2960x; 1.1802x over previous
import functools

import jax
import jax.numpy as jnp
from jax import lax
from jax.experimental import pallas as pl
from jax.experimental.pallas import tpu as pltpu
from jax.experimental.pallas import tpu_sc as plsc

N = 10000
E = 160000
DF = 256
DE = 16
DEO = 64
DH = 512
DG = 64

NP = 10240
NP2 = 10112
K = 5
EC = E // K
NW = 32
EPW = EC // NW
GK = 40
GC = EPW // GK
EPS = EC // 16
SK = 80
SC_CHUNKS = EPS // SK
NC0 = 13

_mesh = plsc.VectorSubcoreMesh(core_axis_name="c", subcore_axis_name="s")

_HI = -65536


def _pack(lo_f32, hi_f32):
    lo = lo_f32.astype(jnp.bfloat16).astype(jnp.float32)
    hi = hi_f32.astype(jnp.bfloat16).astype(jnp.float32)
    lo_b = lax.bitcast_convert_type(lo, jnp.int32)
    hi_b = lax.bitcast_convert_type(hi, jnp.int32)
    return (hi_b & _HI) | lax.shift_right_logical(lo_b, 16)


def _unpack_lo(w):
    return lax.bitcast_convert_type(lax.shift_left(w, 16), jnp.float32)


def _unpack_hi(w):
    return lax.bitcast_convert_type(w & _HI, jnp.float32)


def _pre_body(x, g, gw1, gb1, gw2, gb2, pack_out, g_out):
    xb = x[:]
    pack_out[:] = _pack(xb[:, :128], xb[:, 128:])

    @pl.when(pl.program_id(0) == 0)
    def _():
        h = jnp.maximum(
            jnp.dot(g[:], gw1[:], preferred_element_type=jnp.float32) + gb1[:], 0.0)
        g_out[:] = jnp.dot(h, gw2[:], preferred_element_type=jnp.float32) + gb2[:]


def _pre_call(nodes_p, g, gw1, gb1, gw2, gb2):
    full = lambda shape: pl.BlockSpec(shape, lambda i: (0, 0))
    return pl.pallas_call(
        _pre_body,
        grid=(NP // 256,),
        in_specs=[
            pl.BlockSpec((256, DF), lambda i: (i, 0)),
            full((1, DG)),
            full((DG, DG)),
            full((1, DG)),
            full((DG, DG)),
            full((1, DG)),
        ],
        out_specs=[
            pl.BlockSpec((256, 128), lambda i: (i, 0)),
            full((1, DG)),
        ],
        out_shape=[
            jax.ShapeDtypeStruct((NP, 128), jnp.int32),
            jax.ShapeDtypeStruct((1, DG), jnp.float32),
        ],
    )(nodes_p, g, gw1, gb1, gw2, gb2)


@functools.partial(
    pl.kernel,
    out_type=[
        jax.ShapeDtypeStruct((EC, 128), jnp.int32),
        jax.ShapeDtypeStruct((EC, 128), jnp.int32),
    ],
    mesh=_mesh,
    scratch_types=[
        pltpu.VMEM((GC, GK), jnp.int32),
        pltpu.VMEM((GC, GK), jnp.int32),
        pltpu.VMEM((2, GK, 128), jnp.int32),
        pltpu.VMEM((2, GK, 128), jnp.int32),
        pltpu.SemaphoreType.DMA,
        pltpu.SemaphoreType.DMA,
        pltpu.SemaphoreType.DMA,
        pltpu.SemaphoreType.DMA,
    ],
)
def _gather_kernel(tab, senders3, receivers3,
                   gs_out, gr_out,
                   sidx, ridx, sbuf, rbuf, sem_s0, sem_s1, sem_r0, sem_r1):
    cid = lax.axis_index("c")
    sid = lax.axis_index("s")
    wid = sid * 2 + cid
    base = pl.multiple_of(wid * EPW, 8)
    pltpu.sync_copy(senders3.at[wid], sidx)
    pltpu.sync_copy(receivers3.at[wid], ridx)

    sem_s = (sem_s0, sem_s1)
    sem_r = (sem_r0, sem_r1)

    def start(i):
        p = i % 2
        cs = pltpu.async_copy(tab.at[sidx.at[i]], sbuf.at[p], sem_s[p])
        cr = pltpu.async_copy(tab.at[ridx.at[i]], rbuf.at[p], sem_r[p])
        return cs, cr

    inflight = start(0)
    for i in range(GC):
        p = i % 2
        cs, cr = inflight
        if i + 1 < GC:
            nxt = start(i + 1)
        cs.wait()
        cr.wait()
        if i + 1 < GC:
            inflight = nxt
        off = pl.multiple_of(i * GK, 8)
        pltpu.sync_copy(sbuf.at[p], gs_out.at[pl.ds(base + off, GK)])
        pltpu.sync_copy(rbuf.at[p], gr_out.at[pl.ds(base + off, GK)])


def _edge_body(gs, gr, ea, ws_p, wr_p, w1e, b1, w2, b2, nw1s_p, nw1e, nb1,
               nw2, nb2, ne_out, ma_out, mb_out):
    ws = gs[:]
    s_b = jnp.concatenate(
        [_unpack_lo(ws), _unpack_hi(ws)], axis=1).astype(jnp.bfloat16)
    wr = gr[:]
    r_b = jnp.concatenate(
        [_unpack_lo(wr), _unpack_hi(wr)], axis=1).astype(jnp.bfloat16)
    h1 = jnp.maximum(
        jnp.dot(s_b, ws_p[:], preferred_element_type=jnp.float32)
        + jnp.dot(r_b, wr_p[:], preferred_element_type=jnp.float32)
        + jnp.dot(ea[:].astype(jnp.bfloat16), w1e[:],
                  preferred_element_type=jnp.float32)
        + b1[:], 0.0)
    ne = jnp.dot(h1.astype(jnp.bfloat16), w2[:],
                 preferred_element_type=jnp.float32) + b2[:]
    ne_out[:] = ne
    h2 = jnp.maximum(
        jnp.dot(s_b, nw1s_p[:], preferred_element_type=jnp.float32)
        + jnp.dot(ne.astype(jnp.bfloat16), nw1e[:],
                  preferred_element_type=jnp.float32)
        + nb1[:], 0.0)
    m = jnp.dot(h2.astype(jnp.bfloat16), nw2[:],
                preferred_element_type=jnp.float32) + nb2[:]
    ma_out[:] = m[:, :128]
    mb_out[:] = m[:, 128:]


def _edge_call(gs, gr, ea, ws_p, wr_p, w1e, b1, w2, b2, nw1s_p, nw1e, nb1,
               nw2, nb2):
    BE = 1600
    full = lambda shape: pl.BlockSpec(shape, lambda i: (0, 0))
    return pl.pallas_call(
        _edge_body,
        grid=(EC // BE,),
        in_specs=[
            pl.BlockSpec((BE, 128), lambda i: (i, 0)),
            pl.BlockSpec((BE, 128), lambda i: (i, 0)),
            pl.BlockSpec((BE, DE), lambda i: (i, 0)),
            full((DF, DH)),
            full((DF, DH)),
            full((DE, DH)),
            full((1, DH)),
            full((DH, DEO)),
            full((1, DEO)),
            full((DF, DH)),
            full((DEO, DH)),
            full((1, DH)),
            full((DH, DF)),
            full((1, DF)),
        ],
        out_specs=[
            pl.BlockSpec((BE, DEO), lambda i: (i, 0)),
            pl.BlockSpec((BE, 128), lambda i: (i, 0)),
            pl.BlockSpec((BE, 128), lambda i: (i, 0)),
        ],
        out_shape=[
            jax.ShapeDtypeStruct((EC, DEO), jnp.float32),
            jax.ShapeDtypeStruct((EC, 128), jnp.float32),
            jax.ShapeDtypeStruct((EC, 128), jnp.float32),
        ],
    )(gs, gr, ea, ws_p, wr_p, w1e, b1, w2, b2, nw1s_p, nw1e, nb1, nw2, nb2)


@functools.partial(
    pl.kernel,
    out_type=[
        jax.ShapeDtypeStruct((NP2, 128), jnp.float32),
        jax.ShapeDtypeStruct((NP2, 128), jnp.float32),
    ],
    mesh=_mesh,
    scratch_types=[
        pltpu.VMEM((SC_CHUNKS, SK), jnp.int32),
        pltpu.VMEM((2, SK, 128), jnp.float32),
        pltpu.VMEM((16, 128), jnp.float32),
        pltpu.VMEM_SHARED((NP2, 128), jnp.float32),
        pltpu.SemaphoreType.DMA,
        pltpu.SemaphoreType.DMA,
    ],
)
def _scatter_kernel(msg_a, msg_b, receivers3,
                    sum_a_out, sum_b_out,
                    ridx, mbuf, zbuf, acc, sem_m0, sem_m1):
    cid = lax.axis_index("c")
    sid = lax.axis_index("s")
    ebase = pl.multiple_of(sid * EPS, 8)
    pltpu.sync_copy(receivers3.at[sid], ridx)

    zero = jnp.zeros((16,), jnp.float32)
    for r in range(16):
        for q in range(128 // 16):
            zbuf[r, pl.ds(q * 16, 16)] = zero

    rows = NP2 // 16
    zb = sid * rows

    for j in range(rows // 16):
        pltpu.sync_copy(zbuf, acc.at[pl.ds(zb + j * 16, 16)])
    pltpu.sync_copy(zbuf.at[pl.ds(0, 8)], acc.at[pl.ds(zb + rows - 8, 8)])
    plsc.subcore_barrier()

    sem_m = (sem_m0, sem_m1)

    def accumulate(msg_ref):
        def start(i):
            off = pl.multiple_of(i * SK, 8)
            p = i % 2
            return pltpu.async_copy(
                msg_ref.at[pl.ds(ebase + off, SK)], mbuf.at[p], sem_m[p])

        inflight = start(0)
        for i in range(SC_CHUNKS):
            cp = inflight
            if i + 1 < SC_CHUNKS:
                nxt = start(i + 1)
            cp.wait()
            if i + 1 < SC_CHUNKS:
                inflight = nxt
            pltpu.sync_copy(mbuf.at[i % 2], acc.at[ridx.at[i]], add=True)

    @pl.when(cid == 0)
    def _():
        accumulate(msg_a)

    @pl.when(cid == 1)
    def _():
        accumulate(msg_b)

    plsc.subcore_barrier()

    @pl.when(cid == 0)
    def _():
        pltpu.sync_copy(acc.at[pl.ds(zb, rows)], sum_a_out.at[pl.ds(zb, rows)])

    @pl.when(cid == 1)
    def _():
        pltpu.sync_copy(acc.at[pl.ds(zb, rows)], sum_b_out.at[pl.ds(zb, rows)])


CK = 40
CCH = E // NW // CK


@functools.partial(
    pl.kernel,
    out_type=[
        jax.ShapeDtypeStruct((NP2, 128), jnp.float32),
        jax.ShapeDtypeStruct((NP2, 128), jnp.float32),
    ],
    mesh=_mesh,
    scratch_types=[
        pltpu.VMEM((CCH, CK), jnp.int32),
        pltpu.VMEM((16, 128), jnp.float32),
        pltpu.VMEM((CK, 128), jnp.float32),
        pltpu.VMEM_SHARED((NP2, 128), jnp.float32),
    ],
)
def _count_kernel(receivers3, cnt_a_out, cnt_b_out, ridx, zbuf, ones, acc):
    cid = lax.axis_index("c")
    sid = lax.axis_index("s")
    wid = sid * 2 + cid
    pltpu.sync_copy(receivers3.at[wid], ridx)

    zero = jnp.zeros((16,), jnp.float32)
    one = jnp.ones((16,), jnp.float32)
    for r in range(16):
        for q in range(128 // 16):
            zbuf[r, pl.ds(q * 16, 16)] = zero
    for r in range(CK):
        for q in range(128 // 16):
            ones[r, pl.ds(q * 16, 16)] = one

    rows = NP2 // 16
    zb = sid * rows
    for j in range(rows // 16):
        pltpu.sync_copy(zbuf, acc.at[pl.ds(zb + j * 16, 16)])
    pltpu.sync_copy(zbuf.at[pl.ds(0, 8)], acc.at[pl.ds(zb + rows - 8, 8)])
    plsc.subcore_barrier()

    def cchunk(i, carry):
        pltpu.sync_copy(ones, acc.at[ridx.at[i]], add=True)
        return carry

    lax.fori_loop(0, CCH, cchunk, 0)
    plsc.subcore_barrier()

    @pl.when(cid == 0)
    def _():
        pltpu.sync_copy(acc.at[pl.ds(zb, rows)], cnt_a_out.at[pl.ds(zb, rows)])

    @pl.when(cid == 1)
    def _():
        pltpu.sync_copy(acc.at[pl.ds(zb, rows)], cnt_b_out.at[pl.ds(zb, rows)])


def _div_body(*refs):
    sa_refs = refs[:K]
    sb_refs = refs[K:2 * K]
    cn_refs = refs[2 * K:2 * K + 2]
    o = refs[2 * K + 2]
    sa = sa_refs[0][:]
    sb = sb_refs[0][:]
    for r in sa_refs[1:]:
        sa = sa + r[:]
    for r in sb_refs[1:]:
        sb = sb + r[:]
    cnt = cn_refs[0][:, 0:1] + cn_refs[1][:, 0:1]
    o[:] = jnp.concatenate([sa, sb], axis=1) / jnp.maximum(cnt, 1.0)


def _div_call(sum_as, sum_bs, cnts):
    BR = 632
    block = pl.BlockSpec((BR, 128), lambda i: (i, 0))
    n_in = 2 * K + 2
    return pl.pallas_call(
        _div_body,
        grid=(NP2 // BR,),
        in_specs=[block] * n_in,
        out_specs=pl.BlockSpec((BR, DF), lambda i: (i, 0)),
        out_shape=jax.ShapeDtypeStruct((NP2, DF), jnp.float32),
    )(*sum_as, *sum_bs, *cnts)


def kernel(nodes, senders, receivers, edge_attr, globals, batch,
           eW1, eb1, eW2, eb2, nW1, nb1, nW2, nb2, gW1, gb1, gW2, gb2):
    nodes_p = jnp.pad(nodes, ((0, NP - N), (0, 0)))
    ws_p = eW1[:DF].astype(jnp.bfloat16)
    wr_p = eW1[DF:2 * DF].astype(jnp.bfloat16)
    w1e = eW1[2 * DF:].astype(jnp.bfloat16)
    nw1s_p = nW1[:DF].astype(jnp.bfloat16)
    nw1e = nW1[DF:].astype(jnp.bfloat16)
    eW2b = eW2.astype(jnp.bfloat16)
    nW2b = nW2.astype(jnp.bfloat16)

    senders4 = senders.astype(jnp.int32).reshape(K, NW, GC, GK)
    receivers4 = receivers.astype(jnp.int32).reshape(K, NW, GC, GK)
    receivers4s = receivers.astype(jnp.int32).reshape(K, 16, SC_CHUNKS, SK)
    receivers3c = receivers.astype(jnp.int32).reshape(NW, CCH, CK)

    node_pack, g_out = _pre_call(
        nodes_p, globals.reshape(1, DG),
        gW1, gb1.reshape(1, DG), gW2, gb2.reshape(1, DG))

    eb1r = eb1.reshape(1, DH)
    eb2r = eb2.reshape(1, DEO)
    nb1r = nb1.reshape(1, DH)
    nb2r = nb2.reshape(1, DF)

    ne_parts, sum_as, sum_bs = [], [], []
    g = [None] * K
    g[0] = _gather_kernel(node_pack, senders4[0], receivers4[0])
    cnt_a, cnt_b = _count_kernel(receivers3c)
    for k in range(K):
        if k + 1 < K:
            g[k + 1] = _gather_kernel(
                node_pack, senders4[k + 1], receivers4[k + 1])
        gs, gr = g[k]
        ne_k, ma_k, mb_k = _edge_call(
            gs, gr, edge_attr[k * EC:(k + 1) * EC],
            ws_p, wr_p, w1e, eb1r, eW2b, eb2r, nw1s_p, nw1e, nb1r,
            nW2b, nb2r)
        sa_k, sb_k = _scatter_kernel(ma_k, mb_k, receivers4s[k])
        ne_parts.append(ne_k)
        sum_as.append(sa_k)
        sum_bs.append(sb_k)

    new_nodes = _div_call(sum_as, sum_bs, [cnt_a, cnt_b])[:N]
    new_edge = jnp.concatenate(ne_parts, axis=0)
    return (new_nodes, new_edge, g_out.reshape(DG))
